# Initial kernel scaffold; baseline (speedup 1.0000x reference)
import jax
import jax.numpy as jnp
from jax.experimental import pallas as pl


def kernel(x1, x2, edge_index1, edge_index2, H1, H2, batch1, batch2, W1, b1, W2, b2, W3, b3, Wa, Wt, Wm, bt, Wns, bns, Wfc1, bfc1, Wfc2, bfc2, Wsc, bsc):
    raise NotImplementedError("write your pallas kernel here")


# speedup vs baseline: 20.7773x; 20.7773x over previous
import functools

import jax
import jax.numpy as jnp
from jax import lax
from jax.experimental import pallas as pl
from jax.experimental.pallas import tpu as pltpu
from jax.experimental.pallas import tpu_sc as plsc

N = 4096
NPG = 512
B = 8
E = 262144
F0 = 128
F1 = 128
F2 = 64
F3 = 32
T = 16
_MAPV = [0.0, 0.12, 0.204, 0.186, 0.244, 0.147, 0.039, 0.057]

NCORES = 2
NTILES = 16
NW = NCORES * NTILES
EW = E // NW
CH = 128
NCH = EW // CH
RPT = N // NTILES


def _mesh():
    return plsc.VectorSubcoreMesh(core_axis_name="c", subcore_axis_name="s")


def _make_sc_agg(Fb):
    @functools.partial(
        pl.kernel,
        mesh=_mesh(),
        compiler_params=pltpu.CompilerParams(use_tc_tiling_on_sc=False, needs_layout_passes=False),
        out_type=jax.ShapeDtypeStruct((NCORES, 2, N, Fb), jnp.float32),
        scratch_types=[
            pltpu.VMEM((NCH, CH), jnp.int32),
            pltpu.VMEM((NCH, CH), jnp.int32),
            pltpu.VMEM((CH, Fb), jnp.float32),
            pltpu.VMEM_SHARED((N, Fb), jnp.float32),
            pltpu.VMEM_SHARED((N, Fb), jnp.float32),
            pltpu.SemaphoreType.DMA,
        ],
    )
    def k(h1, h2, src1, dst1, src2, dst2, out, srcv, dstv, rows, agg1, agg2, sem):
        c = lax.axis_index("c")
        s = lax.axis_index("s")
        wid = s * NCORES + c

        def zero_row(j, _):
            for l in range(Fb // 16):
                rows[j, pl.ds(l * 16, 16)] = jnp.zeros((16,), jnp.float32)
            return 0

        lax.fori_loop(0, CH, zero_row, 0)
        for agg in (agg1, agg2):
            for rb in range(RPT // CH):
                pltpu.sync_copy(rows, agg.at[pl.ds(s * RPT + rb * CH, CH)])
        plsc.subcore_barrier()

        for h, sh, dh, agg in ((h1, src1, dst1, agg1), (h2, src2, dst2, agg2)):
            pltpu.sync_copy(sh.at[wid], srcv)
            pltpu.sync_copy(dh.at[wid], dstv)

            def body(ch, _):
                pltpu.async_copy(h.at[srcv.at[ch]], rows, sem).wait()
                pltpu.sync_copy(rows, agg.at[dstv.at[ch]], add=True)
                return 0

            lax.fori_loop(0, NCH, body, 0)
        plsc.subcore_barrier()
        for g, agg in enumerate((agg1, agg2)):
            pltpu.sync_copy(agg.at[pl.ds(s * RPT, RPT)],
                            out.at[c, g, pl.ds(s * RPT, RPT)])

    return k


def _make_sc_cnt():
    @functools.partial(
        pl.kernel,
        mesh=_mesh(),
        compiler_params=pltpu.CompilerParams(use_tc_tiling_on_sc=False, needs_layout_passes=False),
        out_type=jax.ShapeDtypeStruct((NCORES, 2, N, 16), jnp.float32),
        scratch_types=[
            pltpu.VMEM((NCH, CH), jnp.int32),
            pltpu.VMEM((NCH, CH), jnp.int32),
            pltpu.VMEM((CH, 16), jnp.float32),
            pltpu.VMEM_SHARED((N, 16), jnp.float32),
            pltpu.VMEM_SHARED((N, 16), jnp.float32),
        ],
    )
    def k(e1, l1, e2, l2, out, idxv, labv, oh, cnt1, cnt2):
        c = lax.axis_index("c")
        s = lax.axis_index("s")
        wid = s * NCORES + c
        ones16 = jnp.ones((16,), jnp.float32)
        zeros16 = jnp.zeros((16,), jnp.float32)
        iota16 = lax.iota(jnp.int32, 16)

        def zero_row(j, _):
            oh[j, pl.ds(0, 16)] = zeros16
            return 0

        lax.fori_loop(0, CH, zero_row, 0)
        for cnt in (cnt1, cnt2):
            for rb in range(RPT // CH):
                pltpu.sync_copy(oh, cnt.at[pl.ds(s * RPT + rb * CH, CH)])
        plsc.subcore_barrier()

        for e, lb, cnt in ((e1, l1, cnt1), (e2, l2, cnt2)):
            for ept in range(2):
                pltpu.sync_copy(e.at[ept, wid], idxv)
                pltpu.sync_copy(lb.at[ept, wid], labv)

                def body(ch, _):
                    for grp in range(CH // 16):
                        lab = labv[ch, pl.ds(grp * 16, 16)]
                        rowi = iota16 + grp * 16
                        plsc.store_scatter(oh, [rowi, lab], ones16)
                    pltpu.sync_copy(oh, cnt.at[idxv.at[ch]], add=True)
                    for grp in range(CH // 16):
                        lab = labv[ch, pl.ds(grp * 16, 16)]
                        rowi = iota16 + grp * 16
                        plsc.store_scatter(oh, [rowi, lab], zeros16)
                    return 0

                lax.fori_loop(0, NCH, body, 0)
        plsc.subcore_barrier()
        for g, cnt in enumerate((cnt1, cnt2)):
            pltpu.sync_copy(cnt.at[pl.ds(s * RPT, RPT)],
                            out.at[c, g, pl.ds(s * RPT, RPT)])

    return k


_sc_agg144 = _make_sc_agg(F1 + 16)
_sc_agg64 = _make_sc_agg(F2)
_sc_agg32 = _make_sc_agg(F3)
_sc_cnt = _make_sc_cnt()


def _tc_pre(x1, x2, W1, b1):
    def body(x1r, x2r, wr, br, outr):
        pad1 = jnp.ones((N, 1), jnp.float32)
        pad0 = jnp.zeros((N, 15), jnp.float32)
        for g, xr in enumerate((x1r, x2r)):
            h = jnp.dot(xr[...], wr[...], preferred_element_type=jnp.float32) + br[...]
            outr[g] = jnp.concatenate([h, pad1, pad0], axis=1)

    return pl.pallas_call(
        body,
        out_shape=jax.ShapeDtypeStruct((2, N, F1 + 16), jnp.float32),
    )(x1, x2, W1, b1.reshape(1, F1))


def _tc_mid1(part1, hpair, W2, b2):
    def body(pr, hr, wr, br, h2r, degr):
        for g in range(2):
            agg = pr[0, g] + pr[1, g]
            deg = agg[:, F1:F1 + 1]
            g1 = jax.nn.relu(agg[:, :F1] / (deg + 1.0) + hr[g, :, :F1])
            h2r[g] = jnp.dot(g1, wr[...], preferred_element_type=jnp.float32) + br[...]
            degr[g] = deg[:, 0]

    return pl.pallas_call(
        body,
        out_shape=(
            jax.ShapeDtypeStruct((2, N, F2), jnp.float32),
            jax.ShapeDtypeStruct((2, N), jnp.float32),
        ),
    )(part1, hpair, W2, b2.reshape(1, F2))


def _tc_mid2(part2, h2s, deg, W3, b3):
    def body(pr, hr, degr, wr, br, h3r):
        for g in range(2):
            agg = pr[0, g] + pr[1, g]
            g2 = jax.nn.relu(agg / (degr[g][:, None] + 1.0) + hr[g])
            h3r[g] = jnp.dot(g2, wr[...], preferred_element_type=jnp.float32) + br[...]

    return pl.pallas_call(
        body,
        out_shape=jax.ShapeDtypeStruct((2, N, F3), jnp.float32),
    )(part2, h2s, deg, W3, b3.reshape(1, F3))


KC = 128
WNS_CH = (NPG * NPG) // KC


def _tc_tail(part3, h3s, deg, cnt, Wa, Wt, Wm, bt, Wns, bns,
             Wfc1, bfc1, Wfc2, bfc2, Wsc, bsc):
    def body(pr, hr, degr, cntr, war, wtr, wmr, btr, wnsr, bnsr,
             fc1r, bfc1r, fc2r, bfc2r, wscr, bscr, mapr, outr, dv, acc, sc):
        mapv = mapr[...][0]
        c = pl.program_id(0)

        @pl.when(c == 0)
        def _init():
            acc[...] = jnp.zeros((B, F3), jnp.float32)
            for g in range(2):
                agg = pr[0, g] + pr[1, g]
                out3 = agg / (degr[g][:, None] + 1.0) + hr[g]
                dv[g] = jax.nn.relu(out3).reshape(B, NPG, F3)
                sc[g] = jnp.mean(jax.nn.sigmoid(out3))
                cg = cntr[0, g] + cntr[1, g]
                io = lax.broadcasted_iota(jnp.int32, (N, 16), 1)
                nsidx = jnp.max(jnp.where(cg > 0.0, io, 0), axis=1)
                sum_map_h = jnp.sum(cg * mapv[None, :])
                oh = (io == nsidx[:, None]).astype(jnp.float32)
                sum_map_ns = jnp.sum(oh * mapv[None, :])
                sc[2 + g] = (sum_map_h + 2.0 * sum_map_ns) / (2.0 * (E + N))

        a = dv[0, :, pl.ds(c * (NPG // KC), NPG // KC), :]
        d2 = dv[1]
        sec = lax.dot_general(a, d2, (((2,), (2,)), ((0,), (0,))),
                              preferred_element_type=jnp.float32)
        acc[...] += jnp.dot(sec.reshape(B, WNS_CH), wnsr[0],
                            preferred_element_type=jnp.float32)

        @pl.when(c == KC - 1)
        def _final():
            node_scores = jax.nn.sigmoid(acc[...] + bnsr[...])
            ps = []
            for g in range(2):
                d = dv[g]
                xbar = jnp.mean(d, axis=1)
                ctx = jnp.tanh(jnp.dot(xbar, war[...],
                                       preferred_element_type=jnp.float32))
                sig = jax.nn.sigmoid(
                    lax.dot_general(d, ctx, (((2,), (1,)), ((0,), (0,))),
                                    preferred_element_type=jnp.float32))
                p = lax.dot_general(sig, d, (((1,), (1,)), ((0,), (0,))),
                                    preferred_element_type=jnp.float32)
                ps.append(p)
            p1, p2 = ps
            t = jnp.dot(p1, wtr[...].reshape(F3, F3 * T),
                        preferred_element_type=jnp.float32).reshape(B, F3, T)
            scoring = lax.dot_general(t, p2, (((1,), (1,)), ((0,), (0,))),
                                      preferred_element_type=jnp.float32)
            block = jnp.dot(jnp.concatenate([p1, p2], axis=1), wmr[...],
                            preferred_element_type=jnp.float32)
            gl = jax.nn.relu(scoring + block + btr[...])
            s = jnp.concatenate([gl, node_scores], axis=1)
            s = jax.nn.relu(jnp.dot(s, fc1r[...],
                                    preferred_element_type=jnp.float32) + bfc1r[...])
            s = jax.nn.relu(jnp.dot(s, fc2r[...],
                                    preferred_element_type=jnp.float32) + bfc2r[...])
            score = jax.nn.sigmoid(jnp.dot(s, wscr[...],
                                           preferred_element_type=jnp.float32) + bscr[...])
            sup = sc[0] * sc[2] + sc[1] * sc[3]
            outr[...] = jnp.concatenate(
                [score, jnp.full((B, 1), sup, jnp.float32),
                 jnp.zeros((B, 126), jnp.float32)], axis=1)

    const = pl.BlockSpec(lambda c: tuple([0] * 4))
    return pl.pallas_call(
        body,
        grid=(KC,),
        in_specs=[
            pl.BlockSpec((2, 2, N, F3), lambda c: (0, 0, 0, 0)),
            pl.BlockSpec((2, N, F3), lambda c: (0, 0, 0)),
            pl.BlockSpec((2, N), lambda c: (0, 0)),
            pl.BlockSpec((2, 2, N, 16), lambda c: (0, 0, 0, 0)),
            pl.BlockSpec((F3, F3), lambda c: (0, 0)),
            pl.BlockSpec((F3, F3, T), lambda c: (0, 0, 0)),
            pl.BlockSpec((2 * F3, T), lambda c: (0, 0)),
            pl.BlockSpec((1, T), lambda c: (0, 0)),
            pl.BlockSpec((1, WNS_CH, F3), lambda c: (c, 0, 0)),
            pl.BlockSpec((1, F3), lambda c: (0, 0)),
            pl.BlockSpec((T + F3, T), lambda c: (0, 0)),
            pl.BlockSpec((1, T), lambda c: (0, 0)),
            pl.BlockSpec((T, 4), lambda c: (0, 0)),
            pl.BlockSpec((1, 4), lambda c: (0, 0)),
            pl.BlockSpec((4, 1), lambda c: (0, 0)),
            pl.BlockSpec((1, 1), lambda c: (0, 0)),
            pl.BlockSpec((1, 16), lambda c: (0, 0)),
        ],
        out_specs=pl.BlockSpec((B, 128), lambda c: (0, 0)),
        out_shape=jax.ShapeDtypeStruct((B, 128), jnp.float32),
        scratch_shapes=[
            pltpu.VMEM((2, B, NPG, F3), jnp.float32),
            pltpu.VMEM((B, F3), jnp.float32),
            pltpu.SMEM((4,), jnp.float32),
        ],
    )(part3, h3s, deg, cnt, Wa, Wt, Wm, bt.reshape(1, T),
      Wns.reshape(KC, WNS_CH, F3), bns.reshape(1, F3),
      Wfc1, bfc1.reshape(1, T), Wfc2, bfc2.reshape(1, 4),
      Wsc, bsc.reshape(1, 1),
      jnp.array(_MAPV + [0.0] * 8, dtype=jnp.float32).reshape(1, 16))


def kernel(x1, x2, edge_index1, edge_index2, H1, H2, batch1, batch2,
           W1, b1, W2, b2, W3, b3, Wa, Wt, Wm, bt, Wns, bns,
           Wfc1, bfc1, Wfc2, bfc2, Wsc, bsc):
    e1 = edge_index1.reshape(2, NW, NCH, CH)
    e2 = edge_index2.reshape(2, NW, NCH, CH)
    h1l = H1.reshape(2, NW, NCH, CH)
    h2l = H2.reshape(2, NW, NCH, CH)

    cnt = _sc_cnt(e1, h1l, e2, h2l)
    hpair = _tc_pre(x1, x2, W1, b1)
    part1 = _sc_agg144(hpair[0], hpair[1],
                       e1[0], e1[1], e2[0], e2[1])
    h2s, deg = _tc_mid1(part1, hpair, W2, b2)
    part2 = _sc_agg64(h2s[0], h2s[1], e1[0], e1[1], e2[0], e2[1])
    h3s = _tc_mid2(part2, h2s, deg, W3, b3)
    part3 = _sc_agg32(h3s[0], h3s[1], e1[0], e1[1], e2[0], e2[1])
    out = _tc_tail(part3, h3s, deg, cnt, Wa, Wt, Wm, bt, Wns, bns,
                   Wfc1, bfc1, Wfc2, bfc2, Wsc, bsc)
    return out[:, :1], out[0, 1]


# speedup vs baseline: 24.5419x; 1.1812x over previous
import functools

import jax
import jax.numpy as jnp
from jax import lax
from jax.experimental import pallas as pl
from jax.experimental.pallas import tpu as pltpu
from jax.experimental.pallas import tpu_sc as plsc

N = 4096
NPG = 512
B = 8
E = 262144
F0 = 128
F1 = 128
F2 = 64
F3 = 32
T = 16
_MAPV = [0.0, 0.12, 0.204, 0.186, 0.244, 0.147, 0.039, 0.057]

NCORES = 2
NTILES = 16
NW = NCORES * NTILES
EW = E // NW
CH = 128
NCH = EW // CH
RPT = N // NTILES


def _mesh():
    return plsc.VectorSubcoreMesh(core_axis_name="c", subcore_axis_name="s")


def _make_sc_agg(Fb):
    @functools.partial(
        pl.kernel,
        mesh=_mesh(),
        compiler_params=pltpu.CompilerParams(use_tc_tiling_on_sc=False, needs_layout_passes=False),
        out_type=jax.ShapeDtypeStruct((NCORES, 2, N, Fb), jnp.float32),
        scratch_types=[
            pltpu.VMEM((NCH, CH), jnp.int32),
            pltpu.VMEM((NCH, CH), jnp.int32),
            pltpu.VMEM((CH, Fb), jnp.float32),
            pltpu.VMEM((CH, Fb), jnp.float32),
            pltpu.VMEM_SHARED((N, Fb), jnp.float32),
            pltpu.VMEM_SHARED((N, Fb), jnp.float32),
            pltpu.SemaphoreType.DMA,
            pltpu.SemaphoreType.DMA,
        ],
    )
    def k(h1, h2, src1, dst1, src2, dst2, out, srcv, dstv, rows0, rows1,
          agg1, agg2, sem0, sem1):
        rows = rows0
        bufs = (rows0, rows1)
        sems = (sem0, sem1)
        c = lax.axis_index("c")
        s = lax.axis_index("s")
        wid = s * NCORES + c

        def zero_row(j, _):
            for l in range(Fb // 16):
                rows[j, pl.ds(l * 16, 16)] = jnp.zeros((16,), jnp.float32)
            return 0

        lax.fori_loop(0, CH, zero_row, 0)
        for agg in (agg1, agg2):
            for rb in range(RPT // CH):
                pltpu.sync_copy(rows, agg.at[pl.ds(s * RPT + rb * CH, CH)])
        plsc.subcore_barrier()

        for h, sh, dh, agg in ((h1, src1, dst1, agg1), (h2, src2, dst2, agg2)):
            pltpu.sync_copy(sh.at[wid], srcv)
            pltpu.sync_copy(dh.at[wid], dstv)

            hnd = pltpu.async_copy(h.at[srcv.at[0]], bufs[0], sems[0])
            for ch in range(NCH):
                hnd.wait()
                if ch + 1 < NCH:
                    hnd = pltpu.async_copy(h.at[srcv.at[ch + 1]],
                                           bufs[(ch + 1) % 2], sems[(ch + 1) % 2])
                pltpu.sync_copy(bufs[ch % 2], agg.at[dstv.at[ch]], add=True)
        plsc.subcore_barrier()
        for g, agg in enumerate((agg1, agg2)):
            pltpu.sync_copy(agg.at[pl.ds(s * RPT, RPT)],
                            out.at[c, g, pl.ds(s * RPT, RPT)])

    return k


def _make_sc_cnt():
    @functools.partial(
        pl.kernel,
        mesh=_mesh(),
        compiler_params=pltpu.CompilerParams(use_tc_tiling_on_sc=False, needs_layout_passes=False),
        out_type=jax.ShapeDtypeStruct((NCORES, 2, N, 16), jnp.float32),
        scratch_types=[
            pltpu.VMEM((NCH, CH), jnp.int32),
            pltpu.VMEM((NCH, CH), jnp.int32),
            pltpu.VMEM((CH, 16), jnp.float32),
            pltpu.VMEM_SHARED((N, 16), jnp.float32),
            pltpu.VMEM_SHARED((N, 16), jnp.float32),
        ],
    )
    def k(e1, l1, e2, l2, out, idxv, labv, oh, cnt1, cnt2):
        c = lax.axis_index("c")
        s = lax.axis_index("s")
        wid = s * NCORES + c
        ones16 = jnp.ones((16,), jnp.float32)
        zeros16 = jnp.zeros((16,), jnp.float32)
        iota16 = lax.iota(jnp.int32, 16)

        def zero_row(j, _):
            oh[j, pl.ds(0, 16)] = zeros16
            return 0

        lax.fori_loop(0, CH, zero_row, 0)
        for cnt in (cnt1, cnt2):
            for rb in range(RPT // CH):
                pltpu.sync_copy(oh, cnt.at[pl.ds(s * RPT + rb * CH, CH)])
        plsc.subcore_barrier()

        for e, lb, cnt in ((e1, l1, cnt1), (e2, l2, cnt2)):
            for ept in range(2):
                pltpu.sync_copy(e.at[ept, wid], idxv)
                pltpu.sync_copy(lb.at[ept, wid], labv)

                def body(ch, _):
                    for grp in range(CH // 16):
                        lab = labv[ch, pl.ds(grp * 16, 16)]
                        rowi = iota16 + grp * 16
                        plsc.store_scatter(oh, [rowi, lab], ones16)
                    pltpu.sync_copy(oh, cnt.at[idxv.at[ch]], add=True)
                    for grp in range(CH // 16):
                        lab = labv[ch, pl.ds(grp * 16, 16)]
                        rowi = iota16 + grp * 16
                        plsc.store_scatter(oh, [rowi, lab], zeros16)
                    return 0

                lax.fori_loop(0, NCH, body, 0)
        plsc.subcore_barrier()
        for g, cnt in enumerate((cnt1, cnt2)):
            pltpu.sync_copy(cnt.at[pl.ds(s * RPT, RPT)],
                            out.at[c, g, pl.ds(s * RPT, RPT)])

    return k


_sc_agg144 = _make_sc_agg(F1 + 16)
_sc_agg64 = _make_sc_agg(F2)
_sc_agg32 = _make_sc_agg(F3)
_sc_cnt = _make_sc_cnt()


def _tc_pre(x1, x2, W1, b1):
    def body(x1r, x2r, wr, br, outr):
        pad1 = jnp.ones((N, 1), jnp.float32)
        pad0 = jnp.zeros((N, 15), jnp.float32)
        for g, xr in enumerate((x1r, x2r)):
            h = jnp.dot(xr[...], wr[...], preferred_element_type=jnp.float32) + br[...]
            outr[g] = jnp.concatenate([h, pad1, pad0], axis=1)

    return pl.pallas_call(
        body,
        out_shape=jax.ShapeDtypeStruct((2, N, F1 + 16), jnp.float32),
    )(x1, x2, W1, b1.reshape(1, F1))


def _tc_mid1(part1, hpair, W2, b2):
    def body(pr, hr, wr, br, h2r, degr):
        for g in range(2):
            agg = pr[0, g] + pr[1, g]
            deg = agg[:, F1:F1 + 1]
            g1 = jax.nn.relu(agg[:, :F1] / (deg + 1.0) + hr[g, :, :F1])
            h2r[g] = jnp.dot(g1, wr[...], preferred_element_type=jnp.float32) + br[...]
            degr[g] = deg[:, 0]

    return pl.pallas_call(
        body,
        out_shape=(
            jax.ShapeDtypeStruct((2, N, F2), jnp.float32),
            jax.ShapeDtypeStruct((2, N), jnp.float32),
        ),
    )(part1, hpair, W2, b2.reshape(1, F2))


def _tc_mid2(part2, h2s, deg, W3, b3):
    def body(pr, hr, degr, wr, br, h3r):
        for g in range(2):
            agg = pr[0, g] + pr[1, g]
            g2 = jax.nn.relu(agg / (degr[g][:, None] + 1.0) + hr[g])
            h3r[g] = jnp.dot(g2, wr[...], preferred_element_type=jnp.float32) + br[...]

    return pl.pallas_call(
        body,
        out_shape=jax.ShapeDtypeStruct((2, N, F3), jnp.float32),
    )(part2, h2s, deg, W3, b3.reshape(1, F3))


KC = 128
WNS_CH = (NPG * NPG) // KC


def _tc_tail(part3, h3s, deg, cnt, Wa, Wt, Wm, bt, Wns, bns,
             Wfc1, bfc1, Wfc2, bfc2, Wsc, bsc):
    def body(pr, hr, degr, cntr, war, wtr, wmr, btr, wnsr, bnsr,
             fc1r, bfc1r, fc2r, bfc2r, wscr, bscr, mapr, outr, dv, acc, sc):
        mapv = mapr[...][0]
        c = pl.program_id(0)

        @pl.when(c == 0)
        def _init():
            acc[...] = jnp.zeros((B, F3), jnp.float32)
            for g in range(2):
                agg = pr[0, g] + pr[1, g]
                out3 = agg / (degr[g][:, None] + 1.0) + hr[g]
                dv[g] = jax.nn.relu(out3).reshape(B, NPG, F3)
                sc[g] = jnp.mean(jax.nn.sigmoid(out3))
                cg = cntr[0, g] + cntr[1, g]
                io = lax.broadcasted_iota(jnp.int32, (N, 16), 1)
                nsidx = jnp.max(jnp.where(cg > 0.0, io, 0), axis=1)
                sum_map_h = jnp.sum(cg * mapv[None, :])
                oh = (io == nsidx[:, None]).astype(jnp.float32)
                sum_map_ns = jnp.sum(oh * mapv[None, :])
                sc[2 + g] = (sum_map_h + 2.0 * sum_map_ns) / (2.0 * (E + N))

        a = dv[0, :, pl.ds(c * (NPG // KC), NPG // KC), :]
        d2 = dv[1]
        sec = lax.dot_general(a, d2, (((2,), (2,)), ((0,), (0,))),
                              preferred_element_type=jnp.float32)
        acc[...] += jnp.dot(sec.reshape(B, WNS_CH), wnsr[0],
                            preferred_element_type=jnp.float32)

        @pl.when(c == KC - 1)
        def _final():
            node_scores = jax.nn.sigmoid(acc[...] + bnsr[...])
            ps = []
            for g in range(2):
                d = dv[g]
                xbar = jnp.mean(d, axis=1)
                ctx = jnp.tanh(jnp.dot(xbar, war[...],
                                       preferred_element_type=jnp.float32))
                sig = jax.nn.sigmoid(
                    lax.dot_general(d, ctx, (((2,), (1,)), ((0,), (0,))),
                                    preferred_element_type=jnp.float32))
                p = lax.dot_general(sig, d, (((1,), (1,)), ((0,), (0,))),
                                    preferred_element_type=jnp.float32)
                ps.append(p)
            p1, p2 = ps
            t = jnp.dot(p1, wtr[...].reshape(F3, F3 * T),
                        preferred_element_type=jnp.float32).reshape(B, F3, T)
            scoring = lax.dot_general(t, p2, (((1,), (1,)), ((0,), (0,))),
                                      preferred_element_type=jnp.float32)
            block = jnp.dot(jnp.concatenate([p1, p2], axis=1), wmr[...],
                            preferred_element_type=jnp.float32)
            gl = jax.nn.relu(scoring + block + btr[...])
            s = jnp.concatenate([gl, node_scores], axis=1)
            s = jax.nn.relu(jnp.dot(s, fc1r[...],
                                    preferred_element_type=jnp.float32) + bfc1r[...])
            s = jax.nn.relu(jnp.dot(s, fc2r[...],
                                    preferred_element_type=jnp.float32) + bfc2r[...])
            score = jax.nn.sigmoid(jnp.dot(s, wscr[...],
                                           preferred_element_type=jnp.float32) + bscr[...])
            sup = sc[0] * sc[2] + sc[1] * sc[3]
            outr[...] = jnp.concatenate(
                [score, jnp.full((B, 1), sup, jnp.float32),
                 jnp.zeros((B, 126), jnp.float32)], axis=1)

    const = pl.BlockSpec(lambda c: tuple([0] * 4))
    return pl.pallas_call(
        body,
        grid=(KC,),
        in_specs=[
            pl.BlockSpec((2, 2, N, F3), lambda c: (0, 0, 0, 0)),
            pl.BlockSpec((2, N, F3), lambda c: (0, 0, 0)),
            pl.BlockSpec((2, N), lambda c: (0, 0)),
            pl.BlockSpec((2, 2, N, 16), lambda c: (0, 0, 0, 0)),
            pl.BlockSpec((F3, F3), lambda c: (0, 0)),
            pl.BlockSpec((F3, F3, T), lambda c: (0, 0, 0)),
            pl.BlockSpec((2 * F3, T), lambda c: (0, 0)),
            pl.BlockSpec((1, T), lambda c: (0, 0)),
            pl.BlockSpec((1, WNS_CH, F3), lambda c: (c, 0, 0)),
            pl.BlockSpec((1, F3), lambda c: (0, 0)),
            pl.BlockSpec((T + F3, T), lambda c: (0, 0)),
            pl.BlockSpec((1, T), lambda c: (0, 0)),
            pl.BlockSpec((T, 4), lambda c: (0, 0)),
            pl.BlockSpec((1, 4), lambda c: (0, 0)),
            pl.BlockSpec((4, 1), lambda c: (0, 0)),
            pl.BlockSpec((1, 1), lambda c: (0, 0)),
            pl.BlockSpec((1, 16), lambda c: (0, 0)),
        ],
        out_specs=pl.BlockSpec((B, 128), lambda c: (0, 0)),
        out_shape=jax.ShapeDtypeStruct((B, 128), jnp.float32),
        scratch_shapes=[
            pltpu.VMEM((2, B, NPG, F3), jnp.float32),
            pltpu.VMEM((B, F3), jnp.float32),
            pltpu.SMEM((4,), jnp.float32),
        ],
    )(part3, h3s, deg, cnt, Wa, Wt, Wm, bt.reshape(1, T),
      Wns.reshape(KC, WNS_CH, F3), bns.reshape(1, F3),
      Wfc1, bfc1.reshape(1, T), Wfc2, bfc2.reshape(1, 4),
      Wsc, bsc.reshape(1, 1),
      jnp.array(_MAPV + [0.0] * 8, dtype=jnp.float32).reshape(1, 16))


def kernel(x1, x2, edge_index1, edge_index2, H1, H2, batch1, batch2,
           W1, b1, W2, b2, W3, b3, Wa, Wt, Wm, bt, Wns, bns,
           Wfc1, bfc1, Wfc2, bfc2, Wsc, bsc):
    e1 = edge_index1.reshape(2, NW, NCH, CH)
    e2 = edge_index2.reshape(2, NW, NCH, CH)
    h1l = H1.reshape(2, NW, NCH, CH)
    h2l = H2.reshape(2, NW, NCH, CH)

    cnt = _sc_cnt(e1, h1l, e2, h2l)
    hpair = _tc_pre(x1, x2, W1, b1)
    part1 = _sc_agg144(hpair[0], hpair[1],
                       e1[0], e1[1], e2[0], e2[1])
    h2s, deg = _tc_mid1(part1, hpair, W2, b2)
    part2 = _sc_agg64(h2s[0], h2s[1], e1[0], e1[1], e2[0], e2[1])
    h3s = _tc_mid2(part2, h2s, deg, W3, b3)
    part3 = _sc_agg32(h3s[0], h3s[1], e1[0], e1[1], e2[0], e2[1])
    out = _tc_tail(part3, h3s, deg, cnt, Wa, Wt, Wm, bt, Wns, bns,
                   Wfc1, bfc1, Wfc2, bfc2, Wsc, bsc)
    return out[:, :1], out[0, 1]


# speedup vs baseline: 25.0652x; 1.0213x over previous
import functools

import jax
import jax.numpy as jnp
from jax import lax
from jax.experimental import pallas as pl
from jax.experimental.pallas import tpu as pltpu
from jax.experimental.pallas import tpu_sc as plsc

N = 4096
NPG = 512
B = 8
E = 262144
F0 = 128
F1 = 128
F2 = 64
F3 = 32
T = 16
_MAPV = [0.0, 0.12, 0.204, 0.186, 0.244, 0.147, 0.039, 0.057]

NCORES = 2
NTILES = 16
NW = NCORES * NTILES
EW = E // NW
CH = 128
NCH = EW // CH
RPT = N // NTILES


def _mesh():
    return plsc.VectorSubcoreMesh(core_axis_name="c", subcore_axis_name="s")


def _make_sc_agg(Fb):
    @functools.partial(
        pl.kernel,
        mesh=_mesh(),
        compiler_params=pltpu.CompilerParams(use_tc_tiling_on_sc=False, needs_layout_passes=False),
        out_type=jax.ShapeDtypeStruct((NCORES, 2, N, Fb), jnp.float32),
        scratch_types=[
            pltpu.VMEM((NCH, CH), jnp.int32),
            pltpu.VMEM((NCH, CH), jnp.int32),
            pltpu.VMEM((CH, Fb), jnp.float32),
            pltpu.VMEM((CH, Fb), jnp.float32),
            pltpu.VMEM_SHARED((N, Fb), jnp.float32),
            pltpu.VMEM_SHARED((N, Fb), jnp.float32),
            pltpu.SemaphoreType.DMA,
            pltpu.SemaphoreType.DMA,
        ],
    )
    def k(h1, h2, e1, e2, out, srcv, dstv, rows0, rows1,
          agg1, agg2, sem0, sem1):
        rows = rows0
        bufs = (rows0, rows1)
        sems = (sem0, sem1)
        c = lax.axis_index("c")
        s = lax.axis_index("s")
        wid = s * NCORES + c

        def zero_row(j, _):
            for l in range(Fb // 16):
                rows[j, pl.ds(l * 16, 16)] = jnp.zeros((16,), jnp.float32)
            return 0

        lax.fori_loop(0, CH, zero_row, 0)
        for agg in (agg1, agg2):
            for rb in range(RPT // CH):
                pltpu.sync_copy(rows, agg.at[pl.ds(s * RPT + rb * CH, CH)])
        plsc.subcore_barrier()

        for h, e, agg in ((h1, e1, agg1), (h2, e2, agg2)):
            pltpu.sync_copy(e.at[0, wid], srcv)
            pltpu.sync_copy(e.at[1, wid], dstv)

            hnd = pltpu.async_copy(h.at[srcv.at[0]], bufs[0], sems[0])
            for ch in range(NCH):
                hnd.wait()
                if ch + 1 < NCH:
                    hnd = pltpu.async_copy(h.at[srcv.at[ch + 1]],
                                           bufs[(ch + 1) % 2], sems[(ch + 1) % 2])
                pltpu.sync_copy(bufs[ch % 2], agg.at[dstv.at[ch]], add=True)
        plsc.subcore_barrier()
        for g, agg in enumerate((agg1, agg2)):
            pltpu.sync_copy(agg.at[pl.ds(s * RPT, RPT)],
                            out.at[c, g, pl.ds(s * RPT, RPT)])

    return k


def _make_sc_cnt():
    @functools.partial(
        pl.kernel,
        mesh=_mesh(),
        compiler_params=pltpu.CompilerParams(use_tc_tiling_on_sc=False, needs_layout_passes=False),
        out_type=jax.ShapeDtypeStruct((NCORES, 2, N, 16), jnp.float32),
        scratch_types=[
            pltpu.VMEM((NCH, CH), jnp.int32),
            pltpu.VMEM((NCH, CH), jnp.int32),
            pltpu.VMEM((CH, 16), jnp.float32),
            pltpu.VMEM_SHARED((N, 16), jnp.float32),
            pltpu.VMEM_SHARED((N, 16), jnp.float32),
        ],
    )
    def k(e1, l1, e2, l2, out, idxv, labv, oh, cnt1, cnt2):
        c = lax.axis_index("c")
        s = lax.axis_index("s")
        wid = s * NCORES + c
        ones16 = jnp.ones((16,), jnp.float32)
        zeros16 = jnp.zeros((16,), jnp.float32)
        iota16 = lax.iota(jnp.int32, 16)

        def zero_row(j, _):
            oh[j, pl.ds(0, 16)] = zeros16
            return 0

        lax.fori_loop(0, CH, zero_row, 0)
        for cnt in (cnt1, cnt2):
            for rb in range(RPT // CH):
                pltpu.sync_copy(oh, cnt.at[pl.ds(s * RPT + rb * CH, CH)])
        plsc.subcore_barrier()

        for e, lb, cnt in ((e1, l1, cnt1), (e2, l2, cnt2)):
            for ept in range(2):
                pltpu.sync_copy(e.at[ept, wid], idxv)
                pltpu.sync_copy(lb.at[ept, wid], labv)

                def body(ch, _):
                    for grp in range(CH // 16):
                        lab = labv[ch, pl.ds(grp * 16, 16)]
                        rowi = iota16 + grp * 16
                        plsc.store_scatter(oh, [rowi, lab], ones16)
                    pltpu.sync_copy(oh, cnt.at[idxv.at[ch]], add=True)
                    for grp in range(CH // 16):
                        lab = labv[ch, pl.ds(grp * 16, 16)]
                        rowi = iota16 + grp * 16
                        plsc.store_scatter(oh, [rowi, lab], zeros16)
                    return 0

                lax.fori_loop(0, NCH, body, 0)
        plsc.subcore_barrier()
        for g, cnt in enumerate((cnt1, cnt2)):
            pltpu.sync_copy(cnt.at[pl.ds(s * RPT, RPT)],
                            out.at[c, g, pl.ds(s * RPT, RPT)])

    return k


_sc_agg144 = _make_sc_agg(F1 + 16)
_sc_agg64 = _make_sc_agg(F2)
_sc_agg32 = _make_sc_agg(F3)
_sc_cnt = _make_sc_cnt()


def _tc_pre(x1, x2, W1, b1):
    def body(x1r, x2r, wr, br, o1r, o2r):
        pad1 = jnp.ones((N, 1), jnp.float32)
        pad0 = jnp.zeros((N, 15), jnp.float32)
        for xr, outr in ((x1r, o1r), (x2r, o2r)):
            h = jnp.dot(xr[...], wr[...], preferred_element_type=jnp.float32) + br[...]
            outr[...] = jnp.concatenate([h, pad1, pad0], axis=1)

    return pl.pallas_call(
        body,
        out_shape=(
            jax.ShapeDtypeStruct((N, F1 + 16), jnp.float32),
            jax.ShapeDtypeStruct((N, F1 + 16), jnp.float32),
        ),
    )(x1, x2, W1, b1.reshape(1, F1))


def _tc_mid1(part1, h1e, h2e, W2, b2):
    def body(pr, h1r, h2r, wr, br, o1r, o2r, degr):
        for g, (hr, outr) in enumerate(((h1r, o1r), (h2r, o2r))):
            agg = pr[0, g] + pr[1, g]
            deg = agg[:, F1:F1 + 1]
            g1 = jax.nn.relu(agg[:, :F1] / (deg + 1.0) + hr[:, :F1])
            outr[...] = jnp.dot(g1, wr[...], preferred_element_type=jnp.float32) + br[...]
            degr[g] = deg[:, 0]

    return pl.pallas_call(
        body,
        out_shape=(
            jax.ShapeDtypeStruct((N, F2), jnp.float32),
            jax.ShapeDtypeStruct((N, F2), jnp.float32),
            jax.ShapeDtypeStruct((2, N), jnp.float32),
        ),
    )(part1, h1e, h2e, W2, b2.reshape(1, F2))


def _tc_mid2(part2, h2_1, h2_2, deg, W3, b3):
    def body(pr, h1r, h2r, degr, wr, br, o1r, o2r):
        for g, (hr, outr) in enumerate(((h1r, o1r), (h2r, o2r))):
            agg = pr[0, g] + pr[1, g]
            g2 = jax.nn.relu(agg / (degr[g][:, None] + 1.0) + hr[...])
            outr[...] = jnp.dot(g2, wr[...], preferred_element_type=jnp.float32) + br[...]

    return pl.pallas_call(
        body,
        out_shape=(
            jax.ShapeDtypeStruct((N, F3), jnp.float32),
            jax.ShapeDtypeStruct((N, F3), jnp.float32),
        ),
    )(part2, h2_1, h2_2, deg, W3, b3.reshape(1, F3))


KC = 128
WNS_CH = (NPG * NPG) // KC


def _tc_tail(part3, h3_1, h3_2, deg, cnt, Wa, Wt, Wm, bt, Wns, bns,
             Wfc1, bfc1, Wfc2, bfc2, Wsc, bsc):
    def body(pr, h1r, h2r, degr, cntr, war, wtr, wmr, btr, wnsr, bnsr,
             fc1r, bfc1r, fc2r, bfc2r, wscr, bscr, mapr, outr, dv, acc, sc):
        mapv = mapr[...][0]
        hrs = (h1r, h2r)
        c = pl.program_id(0)

        @pl.when(c == 0)
        def _init():
            acc[...] = jnp.zeros((B, F3), jnp.float32)
            for g in range(2):
                agg = pr[0, g] + pr[1, g]
                out3 = agg / (degr[g][:, None] + 1.0) + hrs[g][...]
                dv[g] = jax.nn.relu(out3).reshape(B, NPG, F3)
                sc[g] = jnp.mean(jax.nn.sigmoid(out3))
                cg = cntr[0, g] + cntr[1, g]
                io = lax.broadcasted_iota(jnp.int32, (N, 16), 1)
                nsidx = jnp.max(jnp.where(cg > 0.0, io, 0), axis=1)
                sum_map_h = jnp.sum(cg * mapv[None, :])
                oh = (io == nsidx[:, None]).astype(jnp.float32)
                sum_map_ns = jnp.sum(oh * mapv[None, :])
                sc[2 + g] = (sum_map_h + 2.0 * sum_map_ns) / (2.0 * (E + N))

        a = dv[0, :, pl.ds(c * (NPG // KC), NPG // KC), :]
        d2 = dv[1]
        sec = lax.dot_general(a, d2, (((2,), (2,)), ((0,), (0,))),
                              preferred_element_type=jnp.float32)
        acc[...] += jnp.dot(sec.reshape(B, WNS_CH), wnsr[0],
                            preferred_element_type=jnp.float32)

        @pl.when(c == KC - 1)
        def _final():
            node_scores = jax.nn.sigmoid(acc[...] + bnsr[...])
            ps = []
            for g in range(2):
                d = dv[g]
                xbar = jnp.mean(d, axis=1)
                ctx = jnp.tanh(jnp.dot(xbar, war[...],
                                       preferred_element_type=jnp.float32))
                sig = jax.nn.sigmoid(
                    lax.dot_general(d, ctx, (((2,), (1,)), ((0,), (0,))),
                                    preferred_element_type=jnp.float32))
                p = lax.dot_general(sig, d, (((1,), (1,)), ((0,), (0,))),
                                    preferred_element_type=jnp.float32)
                ps.append(p)
            p1, p2 = ps
            t = jnp.dot(p1, wtr[...].reshape(F3, F3 * T),
                        preferred_element_type=jnp.float32).reshape(B, F3, T)
            scoring = lax.dot_general(t, p2, (((1,), (1,)), ((0,), (0,))),
                                      preferred_element_type=jnp.float32)
            block = jnp.dot(jnp.concatenate([p1, p2], axis=1), wmr[...],
                            preferred_element_type=jnp.float32)
            gl = jax.nn.relu(scoring + block + btr[...])
            s = jnp.concatenate([gl, node_scores], axis=1)
            s = jax.nn.relu(jnp.dot(s, fc1r[...],
                                    preferred_element_type=jnp.float32) + bfc1r[...])
            s = jax.nn.relu(jnp.dot(s, fc2r[...],
                                    preferred_element_type=jnp.float32) + bfc2r[...])
            score = jax.nn.sigmoid(jnp.dot(s, wscr[...],
                                           preferred_element_type=jnp.float32) + bscr[...])
            sup = sc[0] * sc[2] + sc[1] * sc[3]
            outr[...] = jnp.concatenate(
                [score, jnp.full((B, 1), sup, jnp.float32),
                 jnp.zeros((B, 126), jnp.float32)], axis=1)

    const = pl.BlockSpec(lambda c: tuple([0] * 4))
    return pl.pallas_call(
        body,
        grid=(KC,),
        in_specs=[
            pl.BlockSpec((2, 2, N, F3), lambda c: (0, 0, 0, 0)),
            pl.BlockSpec((N, F3), lambda c: (0, 0)),
            pl.BlockSpec((N, F3), lambda c: (0, 0)),
            pl.BlockSpec((2, N), lambda c: (0, 0)),
            pl.BlockSpec((2, 2, N, 16), lambda c: (0, 0, 0, 0)),
            pl.BlockSpec((F3, F3), lambda c: (0, 0)),
            pl.BlockSpec((F3, F3, T), lambda c: (0, 0, 0)),
            pl.BlockSpec((2 * F3, T), lambda c: (0, 0)),
            pl.BlockSpec((1, T), lambda c: (0, 0)),
            pl.BlockSpec((1, WNS_CH, F3), lambda c: (c, 0, 0)),
            pl.BlockSpec((1, F3), lambda c: (0, 0)),
            pl.BlockSpec((T + F3, T), lambda c: (0, 0)),
            pl.BlockSpec((1, T), lambda c: (0, 0)),
            pl.BlockSpec((T, 4), lambda c: (0, 0)),
            pl.BlockSpec((1, 4), lambda c: (0, 0)),
            pl.BlockSpec((4, 1), lambda c: (0, 0)),
            pl.BlockSpec((1, 1), lambda c: (0, 0)),
            pl.BlockSpec((1, 16), lambda c: (0, 0)),
        ],
        out_specs=pl.BlockSpec((B, 128), lambda c: (0, 0)),
        out_shape=jax.ShapeDtypeStruct((B, 128), jnp.float32),
        scratch_shapes=[
            pltpu.VMEM((2, B, NPG, F3), jnp.float32),
            pltpu.VMEM((B, F3), jnp.float32),
            pltpu.SMEM((4,), jnp.float32),
        ],
    )(part3, h3_1, h3_2, deg, cnt, Wa, Wt, Wm, bt.reshape(1, T),
      Wns.reshape(KC, WNS_CH, F3), bns.reshape(1, F3),
      Wfc1, bfc1.reshape(1, T), Wfc2, bfc2.reshape(1, 4),
      Wsc, bsc.reshape(1, 1),
      jnp.array(_MAPV + [0.0] * 8, dtype=jnp.float32).reshape(1, 16))


def kernel(x1, x2, edge_index1, edge_index2, H1, H2, batch1, batch2,
           W1, b1, W2, b2, W3, b3, Wa, Wt, Wm, bt, Wns, bns,
           Wfc1, bfc1, Wfc2, bfc2, Wsc, bsc):
    e1 = edge_index1.reshape(2, NW, NCH, CH)
    e2 = edge_index2.reshape(2, NW, NCH, CH)
    h1l = H1.reshape(2, NW, NCH, CH)
    h2l = H2.reshape(2, NW, NCH, CH)

    cnt = _sc_cnt(e1, h1l, e2, h2l)
    h1e, h2e = _tc_pre(x1, x2, W1, b1)
    part1 = _sc_agg144(h1e, h2e, e1, e2)
    h2_1, h2_2, deg = _tc_mid1(part1, h1e, h2e, W2, b2)
    part2 = _sc_agg64(h2_1, h2_2, e1, e2)
    h3_1, h3_2 = _tc_mid2(part2, h2_1, h2_2, deg, W3, b3)
    part3 = _sc_agg32(h3_1, h3_2, e1, e2)
    out = _tc_tail(part3, h3_1, h3_2, deg, cnt, Wa, Wt, Wm, bt, Wns, bns,
                   Wfc1, bfc1, Wfc2, bfc2, Wsc, bsc)
    return out[:, :1], out[0, 1]


# speedup vs baseline: 25.5785x; 1.0205x over previous
import functools

import jax
import jax.numpy as jnp
from jax import lax
from jax.experimental import pallas as pl
from jax.experimental.pallas import tpu as pltpu
from jax.experimental.pallas import tpu_sc as plsc

N = 4096
NPG = 512
B = 8
E = 262144
F0 = 128
F1 = 128
F2 = 64
F3 = 32
T = 16
_MAPV = [0.0, 0.12, 0.204, 0.186, 0.244, 0.147, 0.039, 0.057]

NCORES = 2
NTILES = 16
NW = NCORES * NTILES
EW = E // NW
CH = 128
NCH = EW // CH
RPT = N // NTILES


def _mesh():
    return plsc.VectorSubcoreMesh(core_axis_name="c", subcore_axis_name="s")


def _make_sc_agg(Fb):
    @functools.partial(
        pl.kernel,
        mesh=_mesh(),
        compiler_params=pltpu.CompilerParams(use_tc_tiling_on_sc=False, needs_layout_passes=False),
        out_type=jax.ShapeDtypeStruct((NCORES, 2, N, Fb), jnp.float32),
        scratch_types=[
            pltpu.VMEM((NCH, CH), jnp.int32),
            pltpu.VMEM((NCH, CH), jnp.int32),
            pltpu.VMEM((CH, Fb), jnp.float32),
            pltpu.VMEM((CH, Fb), jnp.float32),
            pltpu.VMEM_SHARED((N, Fb), jnp.float32),
            pltpu.VMEM_SHARED((N, Fb), jnp.float32),
            pltpu.SemaphoreType.DMA,
            pltpu.SemaphoreType.DMA,
        ],
    )
    def k(h1, h2, e1, e2, out, srcv, dstv, rows0, rows1,
          agg1, agg2, sem0, sem1):
        rows = rows0
        bufs = (rows0, rows1)
        sems = (sem0, sem1)
        c = lax.axis_index("c")
        s = lax.axis_index("s")
        wid = s * NCORES + c

        def zero_row(j, _):
            for l in range(Fb // 16):
                rows[j, pl.ds(l * 16, 16)] = jnp.zeros((16,), jnp.float32)
            return 0

        lax.fori_loop(0, CH, zero_row, 0)
        for agg in (agg1, agg2):
            for rb in range(RPT // CH):
                pltpu.sync_copy(rows, agg.at[pl.ds(s * RPT + rb * CH, CH)])
        plsc.subcore_barrier()

        for h, e, agg in ((h1, e1, agg1), (h2, e2, agg2)):
            pltpu.sync_copy(e.at[0, wid], srcv)
            pltpu.sync_copy(e.at[1, wid], dstv)

            hnd = pltpu.async_copy(h.at[srcv.at[0]], bufs[0], sems[0])
            for ch in range(NCH):
                hnd.wait()
                if ch + 1 < NCH:
                    hnd = pltpu.async_copy(h.at[srcv.at[ch + 1]],
                                           bufs[(ch + 1) % 2], sems[(ch + 1) % 2])
                pltpu.sync_copy(bufs[ch % 2], agg.at[dstv.at[ch]], add=True)
        plsc.subcore_barrier()
        for g, agg in enumerate((agg1, agg2)):
            pltpu.sync_copy(agg.at[pl.ds(s * RPT, RPT)],
                            out.at[c, g, pl.ds(s * RPT, RPT)])

    return k


def _make_sc_cnt():
    @functools.partial(
        pl.kernel,
        mesh=_mesh(),
        compiler_params=pltpu.CompilerParams(use_tc_tiling_on_sc=False, needs_layout_passes=False),
        out_type=jax.ShapeDtypeStruct((NCORES, 2, N, 16), jnp.float32),
        scratch_types=[
            pltpu.VMEM((NCH, CH), jnp.int32),
            pltpu.VMEM((NCH, CH), jnp.int32),
            pltpu.VMEM((CH, 16), jnp.float32),
            pltpu.VMEM_SHARED((N, 16), jnp.float32),
            pltpu.VMEM_SHARED((N, 16), jnp.float32),
        ],
    )
    def k(e1, l1, e2, l2, out, idxv, labv, oh, cnt1, cnt2):
        c = lax.axis_index("c")
        s = lax.axis_index("s")
        wid = s * NCORES + c
        ones16 = jnp.ones((16,), jnp.float32)
        zeros16 = jnp.zeros((16,), jnp.float32)
        iota16 = lax.iota(jnp.int32, 16)

        def zero_row(j, _):
            oh[j, pl.ds(0, 16)] = zeros16
            return 0

        lax.fori_loop(0, CH, zero_row, 0)
        for cnt in (cnt1, cnt2):
            for rb in range(RPT // CH):
                pltpu.sync_copy(oh, cnt.at[pl.ds(s * RPT + rb * CH, CH)])
        plsc.subcore_barrier()

        for e, lb, cnt in ((e1, l1, cnt1), (e2, l2, cnt2)):
            for ept in range(2):
                pltpu.sync_copy(e.at[ept, wid], idxv)
                pltpu.sync_copy(lb.at[ept, wid], labv)

                eights = iota16 * 0 + 8

                def body(ch, _):
                    for grp in range(CH // 16):
                        lab = labv[ch, pl.ds(grp * 16, 16)]
                        rowi = iota16 + grp * 16
                        plsc.store_scatter(oh, [rowi, lab], ones16)
                        if ept == 1:
                            plsc.store_scatter(oh, [rowi, eights], ones16)
                    pltpu.sync_copy(oh, cnt.at[idxv.at[ch]], add=True)
                    for grp in range(CH // 16):
                        lab = labv[ch, pl.ds(grp * 16, 16)]
                        rowi = iota16 + grp * 16
                        plsc.store_scatter(oh, [rowi, lab], zeros16)
                        if ept == 1:
                            plsc.store_scatter(oh, [rowi, eights], zeros16)
                    return 0

                lax.fori_loop(0, NCH, body, 0)
        plsc.subcore_barrier()
        for g, cnt in enumerate((cnt1, cnt2)):
            pltpu.sync_copy(cnt.at[pl.ds(s * RPT, RPT)],
                            out.at[c, g, pl.ds(s * RPT, RPT)])

    return k


_sc_agg128 = _make_sc_agg(F0)
_sc_agg64 = _make_sc_agg(F2)
_sc_agg32 = _make_sc_agg(F3)
_sc_cnt = _make_sc_cnt()


def _tc_mid1(part1, cnt, x1, x2, W1, b1, W2, b2):
    def body(pr, cntr, x1r, x2r, w1r, b1r, w2r, b2r, o1r, o2r, degr):
        for g, (xr, outr) in enumerate(((x1r, o1r), (x2r, o2r))):
            deg = cntr[0, g, :, 8] + cntr[1, g, :, 8]
            degc = deg[:, None]
            h1 = jnp.dot(xr[...], w1r[...], preferred_element_type=jnp.float32) + b1r[...]
            aggx = pr[0, g] + pr[1, g]
            agg1 = (jnp.dot(aggx, w1r[...], preferred_element_type=jnp.float32)
                    + degc * b1r[...])
            g1 = jax.nn.relu(agg1 / (degc + 1.0) + h1)
            outr[...] = jnp.dot(g1, w2r[...], preferred_element_type=jnp.float32) + b2r[...]
            degr[g] = deg

    return pl.pallas_call(
        body,
        out_shape=(
            jax.ShapeDtypeStruct((N, F2), jnp.float32),
            jax.ShapeDtypeStruct((N, F2), jnp.float32),
            jax.ShapeDtypeStruct((2, N), jnp.float32),
        ),
    )(part1, cnt, x1, x2, W1, b1.reshape(1, F1), W2, b2.reshape(1, F2))


def _tc_mid2(part2, h2_1, h2_2, deg, W3, b3):
    def body(pr, h1r, h2r, degr, wr, br, o1r, o2r):
        for g, (hr, outr) in enumerate(((h1r, o1r), (h2r, o2r))):
            agg = pr[0, g] + pr[1, g]
            g2 = jax.nn.relu(agg / (degr[g][:, None] + 1.0) + hr[...])
            outr[...] = jnp.dot(g2, wr[...], preferred_element_type=jnp.float32) + br[...]

    return pl.pallas_call(
        body,
        out_shape=(
            jax.ShapeDtypeStruct((N, F3), jnp.float32),
            jax.ShapeDtypeStruct((N, F3), jnp.float32),
        ),
    )(part2, h2_1, h2_2, deg, W3, b3.reshape(1, F3))


KC = 128
WNS_CH = (NPG * NPG) // KC


def _tc_tail(part3, h3_1, h3_2, deg, cnt, Wa, Wt, Wm, bt, Wns, bns,
             Wfc1, bfc1, Wfc2, bfc2, Wsc, bsc):
    def body(pr, h1r, h2r, degr, cntr, war, wtr, wmr, btr, wnsr, bnsr,
             fc1r, bfc1r, fc2r, bfc2r, wscr, bscr, mapr, outr, dv, acc, sc):
        mapv = mapr[...][0]
        hrs = (h1r, h2r)
        c = pl.program_id(0)

        @pl.when(c == 0)
        def _init():
            acc[...] = jnp.zeros((B, F3), jnp.float32)
            for g in range(2):
                agg = pr[0, g] + pr[1, g]
                out3 = agg / (degr[g][:, None] + 1.0) + hrs[g][...]
                dv[g] = jax.nn.relu(out3).reshape(B, NPG, F3)
                sc[g] = jnp.mean(jax.nn.sigmoid(out3))
                cg = cntr[0, g] + cntr[1, g]
                io = lax.broadcasted_iota(jnp.int32, (N, 16), 1)
                nsidx = jnp.max(jnp.where((cg > 0.0) & (io < 8), io, 0), axis=1)
                sum_map_h = jnp.sum(cg * mapv[None, :])
                oh = (io == nsidx[:, None]).astype(jnp.float32)
                sum_map_ns = jnp.sum(oh * mapv[None, :])
                sc[2 + g] = (sum_map_h + 2.0 * sum_map_ns) / (2.0 * (E + N))

        a = dv[0, :, pl.ds(c * (NPG // KC), NPG // KC), :]
        d2 = dv[1]
        sec = lax.dot_general(a, d2, (((2,), (2,)), ((0,), (0,))),
                              preferred_element_type=jnp.float32)
        acc[...] += jnp.dot(sec.reshape(B, WNS_CH), wnsr[...],
                            preferred_element_type=jnp.float32)

        @pl.when(c == KC - 1)
        def _final():
            node_scores = jax.nn.sigmoid(acc[...] + bnsr[...])
            ps = []
            for g in range(2):
                d = dv[g]
                xbar = jnp.mean(d, axis=1)
                ctx = jnp.tanh(jnp.dot(xbar, war[...],
                                       preferred_element_type=jnp.float32))
                sig = jax.nn.sigmoid(
                    lax.dot_general(d, ctx, (((2,), (1,)), ((0,), (0,))),
                                    preferred_element_type=jnp.float32))
                p = lax.dot_general(sig, d, (((1,), (1,)), ((0,), (0,))),
                                    preferred_element_type=jnp.float32)
                ps.append(p)
            p1, p2 = ps
            t = jnp.dot(p1, wtr[...].reshape(F3, F3 * T),
                        preferred_element_type=jnp.float32).reshape(B, F3, T)
            scoring = lax.dot_general(t, p2, (((1,), (1,)), ((0,), (0,))),
                                      preferred_element_type=jnp.float32)
            block = jnp.dot(jnp.concatenate([p1, p2], axis=1), wmr[...],
                            preferred_element_type=jnp.float32)
            gl = jax.nn.relu(scoring + block + btr[...])
            s = jnp.concatenate([gl, node_scores], axis=1)
            s = jax.nn.relu(jnp.dot(s, fc1r[...],
                                    preferred_element_type=jnp.float32) + bfc1r[...])
            s = jax.nn.relu(jnp.dot(s, fc2r[...],
                                    preferred_element_type=jnp.float32) + bfc2r[...])
            score = jax.nn.sigmoid(jnp.dot(s, wscr[...],
                                           preferred_element_type=jnp.float32) + bscr[...])
            sup = sc[0] * sc[2] + sc[1] * sc[3]
            outr[...] = jnp.concatenate(
                [score, jnp.full((B, 1), sup, jnp.float32),
                 jnp.zeros((B, 126), jnp.float32)], axis=1)

    const = pl.BlockSpec(lambda c: tuple([0] * 4))
    return pl.pallas_call(
        body,
        grid=(KC,),
        in_specs=[
            pl.BlockSpec((2, 2, N, F3), lambda c: (0, 0, 0, 0)),
            pl.BlockSpec((N, F3), lambda c: (0, 0)),
            pl.BlockSpec((N, F3), lambda c: (0, 0)),
            pl.BlockSpec((2, N), lambda c: (0, 0)),
            pl.BlockSpec((2, 2, N, 16), lambda c: (0, 0, 0, 0)),
            pl.BlockSpec((F3, F3), lambda c: (0, 0)),
            pl.BlockSpec((F3, F3, T), lambda c: (0, 0, 0)),
            pl.BlockSpec((2 * F3, T), lambda c: (0, 0)),
            pl.BlockSpec((1, T), lambda c: (0, 0)),
            pl.BlockSpec((WNS_CH, F3), lambda c: (c, 0)),
            pl.BlockSpec((1, F3), lambda c: (0, 0)),
            pl.BlockSpec((T + F3, T), lambda c: (0, 0)),
            pl.BlockSpec((1, T), lambda c: (0, 0)),
            pl.BlockSpec((T, 4), lambda c: (0, 0)),
            pl.BlockSpec((1, 4), lambda c: (0, 0)),
            pl.BlockSpec((4, 1), lambda c: (0, 0)),
            pl.BlockSpec((1, 1), lambda c: (0, 0)),
            pl.BlockSpec((1, 16), lambda c: (0, 0)),
        ],
        out_specs=pl.BlockSpec((B, 128), lambda c: (0, 0)),
        out_shape=jax.ShapeDtypeStruct((B, 128), jnp.float32),
        scratch_shapes=[
            pltpu.VMEM((2, B, NPG, F3), jnp.float32),
            pltpu.VMEM((B, F3), jnp.float32),
            pltpu.SMEM((4,), jnp.float32),
        ],
    )(part3, h3_1, h3_2, deg, cnt, Wa, Wt, Wm, bt.reshape(1, T),
      Wns, bns.reshape(1, F3),
      Wfc1, bfc1.reshape(1, T), Wfc2, bfc2.reshape(1, 4),
      Wsc, bsc.reshape(1, 1),
      jnp.array(_MAPV + [0.0] * 8, dtype=jnp.float32).reshape(1, 16))


def kernel(x1, x2, edge_index1, edge_index2, H1, H2, batch1, batch2,
           W1, b1, W2, b2, W3, b3, Wa, Wt, Wm, bt, Wns, bns,
           Wfc1, bfc1, Wfc2, bfc2, Wsc, bsc):
    e1 = edge_index1.reshape(2, NW, NCH, CH)
    e2 = edge_index2.reshape(2, NW, NCH, CH)
    h1l = H1.reshape(2, NW, NCH, CH)
    h2l = H2.reshape(2, NW, NCH, CH)

    cnt = _sc_cnt(e1, h1l, e2, h2l)
    part1 = _sc_agg128(x1, x2, e1, e2)
    h2_1, h2_2, deg = _tc_mid1(part1, cnt, x1, x2, W1, b1, W2, b2)
    part2 = _sc_agg64(h2_1, h2_2, e1, e2)
    h3_1, h3_2 = _tc_mid2(part2, h2_1, h2_2, deg, W3, b3)
    part3 = _sc_agg32(h3_1, h3_2, e1, e2)
    out = _tc_tail(part3, h3_1, h3_2, deg, cnt, Wa, Wt, Wm, bt, Wns, bns,
                   Wfc1, bfc1, Wfc2, bfc2, Wsc, bsc)
    return out[:, :1], out[0, 1]


# speedup vs baseline: 29.6492x; 1.1591x over previous
import functools

import jax
import jax.numpy as jnp
from jax import lax
from jax.experimental import pallas as pl
from jax.experimental.pallas import tpu as pltpu
from jax.experimental.pallas import tpu_sc as plsc

N = 4096
NPG = 512
B = 8
E = 262144
F0 = 128
F1 = 128
F2 = 64
F3 = 32
T = 16
_MAPV = [0.0, 0.12, 0.204, 0.186, 0.244, 0.147, 0.039, 0.057]

NCORES = 2
NTILES = 16
NW = NCORES * NTILES
EW = E // NW
CH = 128
NCH = EW // CH
RPT = N // NTILES


def _mesh():
    return plsc.VectorSubcoreMesh(core_axis_name="c", subcore_axis_name="s")


def _make_sc_agg(Fb):
    @functools.partial(
        pl.kernel,
        mesh=_mesh(),
        compiler_params=pltpu.CompilerParams(use_tc_tiling_on_sc=False, needs_layout_passes=False),
        out_type=jax.ShapeDtypeStruct((NCORES, 2, N, Fb), jnp.float32),
        scratch_types=[
            pltpu.VMEM((NCH, CH), jnp.int32),
            pltpu.VMEM((NCH, CH), jnp.int32),
            pltpu.VMEM((CH, Fb), jnp.float32),
            pltpu.VMEM((CH, Fb), jnp.float32),
            pltpu.VMEM((CH, Fb), jnp.float32),
            pltpu.VMEM((CH, Fb), jnp.float32),
            pltpu.VMEM_SHARED((N, Fb), jnp.float32),
            pltpu.SemaphoreType.DMA,
            pltpu.SemaphoreType.DMA,
            pltpu.SemaphoreType.DMA,
            pltpu.SemaphoreType.DMA,
            pltpu.SemaphoreType.DMA,
            pltpu.SemaphoreType.DMA,
            pltpu.SemaphoreType.DMA,
            pltpu.SemaphoreType.DMA,
        ],
    )
    def k(h1, h2, e1, e2, out, srcv, dstv, rows0, rows1, rows2, rows3,
          agg, gs0, gs1, gs2, gs3, ss0, ss1, ss2, ss3):
        rows = rows0
        bufs = (rows0, rows1, rows2, rows3)
        gsems = (gs0, gs1, gs2, gs3)
        ssems = (ss0, ss1, ss2, ss3)
        c = lax.axis_index("c")
        s = lax.axis_index("s")
        wid = s * NCORES + c

        def zero_agg():
            def zero_row(j, _):
                for l in range(Fb // 16):
                    rows[j, pl.ds(l * 16, 16)] = jnp.zeros((16,), jnp.float32)
                return 0

            lax.fori_loop(0, CH, zero_row, 0)
            for rb in range(RPT // CH):
                pltpu.sync_copy(rows, agg.at[pl.ds(s * RPT + rb * CH, CH)])
            plsc.subcore_barrier()

        zero_agg()
        for g, (h, e) in enumerate(((h1, e1), (h2, e2))):
            pltpu.sync_copy(e.at[0, wid], srcv)
            pltpu.sync_copy(e.at[1, wid], dstv)

            gh = [None] * NCH
            sh = [None] * NCH
            gh[0] = pltpu.async_copy(h.at[srcv.at[0]], bufs[0], gsems[0])
            gh[1] = pltpu.async_copy(h.at[srcv.at[1]], bufs[1], gsems[1])
            for ch in range(NCH):
                b = ch % 4
                gh[ch].wait()
                sh[ch] = pltpu.async_copy(bufs[b], agg.at[dstv.at[ch]],
                                          ssems[b], add=True)
                if ch >= 2:
                    sh[ch - 2].wait()
                if ch + 2 < NCH:
                    nb = (ch + 2) % 4
                    gh[ch + 2] = pltpu.async_copy(h.at[srcv.at[ch + 2]],
                                                  bufs[nb], gsems[nb])
            sh[NCH - 2].wait()
            sh[NCH - 1].wait()
            plsc.subcore_barrier()
            pltpu.sync_copy(agg.at[pl.ds(s * RPT, RPT)],
                            out.at[c, g, pl.ds(s * RPT, RPT)])
            if g == 0:
                zero_agg()

    return k


def _make_sc_cnt():
    @functools.partial(
        pl.kernel,
        mesh=_mesh(),
        compiler_params=pltpu.CompilerParams(use_tc_tiling_on_sc=False, needs_layout_passes=False),
        out_type=jax.ShapeDtypeStruct((NCORES, 2, N, 16), jnp.float32),
        scratch_types=[
            pltpu.VMEM((NCH, CH), jnp.int32),
            pltpu.VMEM((NCH, CH), jnp.int32),
            pltpu.VMEM((CH, 16), jnp.float32),
            pltpu.VMEM_SHARED((N, 16), jnp.float32),
            pltpu.VMEM_SHARED((N, 16), jnp.float32),
        ],
    )
    def k(e1, l1, e2, l2, out, idxv, labv, oh, cnt1, cnt2):
        c = lax.axis_index("c")
        s = lax.axis_index("s")
        wid = s * NCORES + c
        ones16 = jnp.ones((16,), jnp.float32)
        zeros16 = jnp.zeros((16,), jnp.float32)
        iota16 = lax.iota(jnp.int32, 16)

        def zero_row(j, _):
            oh[j, pl.ds(0, 16)] = zeros16
            return 0

        lax.fori_loop(0, CH, zero_row, 0)
        for cnt in (cnt1, cnt2):
            for rb in range(RPT // CH):
                pltpu.sync_copy(oh, cnt.at[pl.ds(s * RPT + rb * CH, CH)])
        plsc.subcore_barrier()

        for e, lb, cnt in ((e1, l1, cnt1), (e2, l2, cnt2)):
            for ept in range(2):
                pltpu.sync_copy(e.at[ept, wid], idxv)
                pltpu.sync_copy(lb.at[ept, wid], labv)

                eights = iota16 * 0 + 8

                def body(ch, _):
                    for grp in range(CH // 16):
                        lab = labv[ch, pl.ds(grp * 16, 16)]
                        rowi = iota16 + grp * 16
                        plsc.store_scatter(oh, [rowi, lab], ones16)
                        if ept == 1:
                            plsc.store_scatter(oh, [rowi, eights], ones16)
                    pltpu.sync_copy(oh, cnt.at[idxv.at[ch]], add=True)
                    for grp in range(CH // 16):
                        lab = labv[ch, pl.ds(grp * 16, 16)]
                        rowi = iota16 + grp * 16
                        plsc.store_scatter(oh, [rowi, lab], zeros16)
                        if ept == 1:
                            plsc.store_scatter(oh, [rowi, eights], zeros16)
                    return 0

                lax.fori_loop(0, NCH, body, 0)
        plsc.subcore_barrier()
        for g, cnt in enumerate((cnt1, cnt2)):
            pltpu.sync_copy(cnt.at[pl.ds(s * RPT, RPT)],
                            out.at[c, g, pl.ds(s * RPT, RPT)])

    return k


_sc_agg128 = _make_sc_agg(F0)
_sc_agg64 = _make_sc_agg(F2)
_sc_agg32 = _make_sc_agg(F3)
_sc_cnt = _make_sc_cnt()


def _tc_mid1(part1, cnt, x1, x2, W1, b1, W2, b2):
    def body(pr, cntr, x1r, x2r, w1r, b1r, w2r, b2r, o1r, o2r, degr):
        for g, (xr, outr) in enumerate(((x1r, o1r), (x2r, o2r))):
            deg = cntr[0, g, :, 8] + cntr[1, g, :, 8]
            degc = deg[:, None]
            h1 = jnp.dot(xr[...], w1r[...], preferred_element_type=jnp.float32) + b1r[...]
            aggx = pr[0, g] + pr[1, g]
            agg1 = (jnp.dot(aggx, w1r[...], preferred_element_type=jnp.float32)
                    + degc * b1r[...])
            g1 = jax.nn.relu(agg1 / (degc + 1.0) + h1)
            outr[...] = jnp.dot(g1, w2r[...], preferred_element_type=jnp.float32) + b2r[...]
            degr[g] = deg

    return pl.pallas_call(
        body,
        out_shape=(
            jax.ShapeDtypeStruct((N, F2), jnp.float32),
            jax.ShapeDtypeStruct((N, F2), jnp.float32),
            jax.ShapeDtypeStruct((2, N), jnp.float32),
        ),
    )(part1, cnt, x1, x2, W1, b1.reshape(1, F1), W2, b2.reshape(1, F2))


def _tc_mid2(part2, h2_1, h2_2, deg, W3, b3):
    def body(pr, h1r, h2r, degr, wr, br, o1r, o2r):
        for g, (hr, outr) in enumerate(((h1r, o1r), (h2r, o2r))):
            agg = pr[0, g] + pr[1, g]
            g2 = jax.nn.relu(agg / (degr[g][:, None] + 1.0) + hr[...])
            outr[...] = jnp.dot(g2, wr[...], preferred_element_type=jnp.float32) + br[...]

    return pl.pallas_call(
        body,
        out_shape=(
            jax.ShapeDtypeStruct((N, F3), jnp.float32),
            jax.ShapeDtypeStruct((N, F3), jnp.float32),
        ),
    )(part2, h2_1, h2_2, deg, W3, b3.reshape(1, F3))


KC = 128
WNS_CH = (NPG * NPG) // KC


def _tc_tail(part3, h3_1, h3_2, deg, cnt, Wa, Wt, Wm, bt, Wns, bns,
             Wfc1, bfc1, Wfc2, bfc2, Wsc, bsc):
    def body(pr, h1r, h2r, degr, cntr, war, wtr, wmr, btr, wnsr, bnsr,
             fc1r, bfc1r, fc2r, bfc2r, wscr, bscr, mapr, outr, dv, acc, sc):
        mapv = mapr[...][0]
        hrs = (h1r, h2r)
        c = pl.program_id(0)

        @pl.when(c == 0)
        def _init():
            acc[...] = jnp.zeros((B, F3), jnp.float32)
            for g in range(2):
                agg = pr[0, g] + pr[1, g]
                out3 = agg / (degr[g][:, None] + 1.0) + hrs[g][...]
                dv[g] = jax.nn.relu(out3).reshape(B, NPG, F3)
                sc[g] = jnp.mean(jax.nn.sigmoid(out3))
                cg = cntr[0, g] + cntr[1, g]
                io = lax.broadcasted_iota(jnp.int32, (N, 16), 1)
                nsidx = jnp.max(jnp.where((cg > 0.0) & (io < 8), io, 0), axis=1)
                sum_map_h = jnp.sum(cg * mapv[None, :])
                oh = (io == nsidx[:, None]).astype(jnp.float32)
                sum_map_ns = jnp.sum(oh * mapv[None, :])
                sc[2 + g] = (sum_map_h + 2.0 * sum_map_ns) / (2.0 * (E + N))

        a = dv[0, :, pl.ds(c * (NPG // KC), NPG // KC), :]
        d2 = dv[1]
        sec = lax.dot_general(a, d2, (((2,), (2,)), ((0,), (0,))),
                              preferred_element_type=jnp.float32)
        acc[...] += jnp.dot(sec.reshape(B, WNS_CH), wnsr[...],
                            preferred_element_type=jnp.float32)

        @pl.when(c == KC - 1)
        def _final():
            node_scores = jax.nn.sigmoid(acc[...] + bnsr[...])
            ps = []
            for g in range(2):
                d = dv[g]
                xbar = jnp.mean(d, axis=1)
                ctx = jnp.tanh(jnp.dot(xbar, war[...],
                                       preferred_element_type=jnp.float32))
                sig = jax.nn.sigmoid(
                    lax.dot_general(d, ctx, (((2,), (1,)), ((0,), (0,))),
                                    preferred_element_type=jnp.float32))
                p = lax.dot_general(sig, d, (((1,), (1,)), ((0,), (0,))),
                                    preferred_element_type=jnp.float32)
                ps.append(p)
            p1, p2 = ps
            t = jnp.dot(p1, wtr[...].reshape(F3, F3 * T),
                        preferred_element_type=jnp.float32).reshape(B, F3, T)
            scoring = lax.dot_general(t, p2, (((1,), (1,)), ((0,), (0,))),
                                      preferred_element_type=jnp.float32)
            block = jnp.dot(jnp.concatenate([p1, p2], axis=1), wmr[...],
                            preferred_element_type=jnp.float32)
            gl = jax.nn.relu(scoring + block + btr[...])
            s = jnp.concatenate([gl, node_scores], axis=1)
            s = jax.nn.relu(jnp.dot(s, fc1r[...],
                                    preferred_element_type=jnp.float32) + bfc1r[...])
            s = jax.nn.relu(jnp.dot(s, fc2r[...],
                                    preferred_element_type=jnp.float32) + bfc2r[...])
            score = jax.nn.sigmoid(jnp.dot(s, wscr[...],
                                           preferred_element_type=jnp.float32) + bscr[...])
            sup = sc[0] * sc[2] + sc[1] * sc[3]
            outr[...] = jnp.concatenate(
                [score, jnp.full((B, 1), sup, jnp.float32),
                 jnp.zeros((B, 126), jnp.float32)], axis=1)

    const = pl.BlockSpec(lambda c: tuple([0] * 4))
    return pl.pallas_call(
        body,
        grid=(KC,),
        in_specs=[
            pl.BlockSpec((2, 2, N, F3), lambda c: (0, 0, 0, 0)),
            pl.BlockSpec((N, F3), lambda c: (0, 0)),
            pl.BlockSpec((N, F3), lambda c: (0, 0)),
            pl.BlockSpec((2, N), lambda c: (0, 0)),
            pl.BlockSpec((2, 2, N, 16), lambda c: (0, 0, 0, 0)),
            pl.BlockSpec((F3, F3), lambda c: (0, 0)),
            pl.BlockSpec((F3, F3, T), lambda c: (0, 0, 0)),
            pl.BlockSpec((2 * F3, T), lambda c: (0, 0)),
            pl.BlockSpec((1, T), lambda c: (0, 0)),
            pl.BlockSpec((WNS_CH, F3), lambda c: (c, 0)),
            pl.BlockSpec((1, F3), lambda c: (0, 0)),
            pl.BlockSpec((T + F3, T), lambda c: (0, 0)),
            pl.BlockSpec((1, T), lambda c: (0, 0)),
            pl.BlockSpec((T, 4), lambda c: (0, 0)),
            pl.BlockSpec((1, 4), lambda c: (0, 0)),
            pl.BlockSpec((4, 1), lambda c: (0, 0)),
            pl.BlockSpec((1, 1), lambda c: (0, 0)),
            pl.BlockSpec((1, 16), lambda c: (0, 0)),
        ],
        out_specs=pl.BlockSpec((B, 128), lambda c: (0, 0)),
        out_shape=jax.ShapeDtypeStruct((B, 128), jnp.float32),
        scratch_shapes=[
            pltpu.VMEM((2, B, NPG, F3), jnp.float32),
            pltpu.VMEM((B, F3), jnp.float32),
            pltpu.SMEM((4,), jnp.float32),
        ],
    )(part3, h3_1, h3_2, deg, cnt, Wa, Wt, Wm, bt.reshape(1, T),
      Wns, bns.reshape(1, F3),
      Wfc1, bfc1.reshape(1, T), Wfc2, bfc2.reshape(1, 4),
      Wsc, bsc.reshape(1, 1),
      jnp.array(_MAPV + [0.0] * 8, dtype=jnp.float32).reshape(1, 16))


def kernel(x1, x2, edge_index1, edge_index2, H1, H2, batch1, batch2,
           W1, b1, W2, b2, W3, b3, Wa, Wt, Wm, bt, Wns, bns,
           Wfc1, bfc1, Wfc2, bfc2, Wsc, bsc):
    e1 = edge_index1.reshape(2, NW, NCH, CH)
    e2 = edge_index2.reshape(2, NW, NCH, CH)
    h1l = H1.reshape(2, NW, NCH, CH)
    h2l = H2.reshape(2, NW, NCH, CH)

    cnt = _sc_cnt(e1, h1l, e2, h2l)
    part1 = _sc_agg128(x1, x2, e1, e2)
    h2_1, h2_2, deg = _tc_mid1(part1, cnt, x1, x2, W1, b1, W2, b2)
    part2 = _sc_agg64(h2_1, h2_2, e1, e2)
    h3_1, h3_2 = _tc_mid2(part2, h2_1, h2_2, deg, W3, b3)
    part3 = _sc_agg32(h3_1, h3_2, e1, e2)
    out = _tc_tail(part3, h3_1, h3_2, deg, cnt, Wa, Wt, Wm, bt, Wns, bns,
                   Wfc1, bfc1, Wfc2, bfc2, Wsc, bsc)
    return out[:, :1], out[0, 1]


# speedup vs baseline: 30.9961x; 1.0454x over previous
import functools

import jax
import jax.numpy as jnp
from jax import lax
from jax.experimental import pallas as pl
from jax.experimental.pallas import tpu as pltpu
from jax.experimental.pallas import tpu_sc as plsc

N = 4096
NPG = 512
B = 8
E = 262144
F0 = 128
F1 = 128
F2 = 64
F3 = 32
T = 16
_MAPV = [0.0, 0.12, 0.204, 0.186, 0.244, 0.147, 0.039, 0.057]

NCORES = 2
NTILES = 16
NW = NCORES * NTILES
EW = E // NW
CH = 128
NCH = EW // CH
RPT = N // NTILES


def _mesh():
    return plsc.VectorSubcoreMesh(core_axis_name="c", subcore_axis_name="s")


def _make_sc_agg(Fb):
    @functools.partial(
        pl.kernel,
        mesh=_mesh(),
        compiler_params=pltpu.CompilerParams(use_tc_tiling_on_sc=False, needs_layout_passes=False),
        out_type=jax.ShapeDtypeStruct((NCORES, 2, N, Fb), jnp.float32),
        scratch_types=[
            pltpu.VMEM((NCH, CH), jnp.int32),
            pltpu.VMEM((NCH, CH), jnp.int32),
            pltpu.VMEM((CH, Fb), jnp.float32),
            pltpu.VMEM((CH, Fb), jnp.float32),
            pltpu.VMEM((CH, Fb), jnp.float32),
            pltpu.VMEM((CH, Fb), jnp.float32),
            pltpu.VMEM_SHARED((N, Fb), jnp.float32),
            pltpu.SemaphoreType.DMA,
            pltpu.SemaphoreType.DMA,
            pltpu.SemaphoreType.DMA,
            pltpu.SemaphoreType.DMA,
            pltpu.SemaphoreType.DMA,
            pltpu.SemaphoreType.DMA,
            pltpu.SemaphoreType.DMA,
            pltpu.SemaphoreType.DMA,
        ],
    )
    def k(h1, h2, e1, e2, out, srcv, dstv, rows0, rows1, rows2, rows3,
          agg, gs0, gs1, gs2, gs3, ss0, ss1, ss2, ss3):
        rows = rows0
        bufs = (rows0, rows1, rows2, rows3)
        gsems = (gs0, gs1, gs2, gs3)
        ssems = (ss0, ss1, ss2, ss3)
        c = lax.axis_index("c")
        s = lax.axis_index("s")
        wid = s * NCORES + c

        def zero_agg():
            def zero_row(j, _):
                for l in range(Fb // 16):
                    rows[j, pl.ds(l * 16, 16)] = jnp.zeros((16,), jnp.float32)
                return 0

            lax.fori_loop(0, CH, zero_row, 0)
            for rb in range(RPT // CH):
                pltpu.sync_copy(rows, agg.at[pl.ds(s * RPT + rb * CH, CH)])
            plsc.subcore_barrier()

        zero_agg()
        for g, (h, e) in enumerate(((h1, e1), (h2, e2))):
            pltpu.sync_copy(e.at[0, wid], srcv)
            pltpu.sync_copy(e.at[1, wid], dstv)

            gh = [None] * NCH
            sh = [None] * NCH
            gh[0] = pltpu.async_copy(h.at[srcv.at[0]], bufs[0], gsems[0])
            gh[1] = pltpu.async_copy(h.at[srcv.at[1]], bufs[1], gsems[1])
            for ch in range(NCH):
                b = ch % 4
                gh[ch].wait()
                sh[ch] = pltpu.async_copy(bufs[b], agg.at[dstv.at[ch]],
                                          ssems[b], add=True)
                if ch >= 2:
                    sh[ch - 2].wait()
                if ch + 2 < NCH:
                    nb = (ch + 2) % 4
                    gh[ch + 2] = pltpu.async_copy(h.at[srcv.at[ch + 2]],
                                                  bufs[nb], gsems[nb])
            sh[NCH - 2].wait()
            sh[NCH - 1].wait()
            plsc.subcore_barrier()
            pltpu.sync_copy(agg.at[pl.ds(s * RPT, RPT)],
                            out.at[c, g, pl.ds(s * RPT, RPT)])
            if g == 0:
                zero_agg()

    return k


def _make_sc_cnt():
    @functools.partial(
        pl.kernel,
        mesh=_mesh(),
        compiler_params=pltpu.CompilerParams(use_tc_tiling_on_sc=False, needs_layout_passes=False),
        out_type=jax.ShapeDtypeStruct((NCORES, 2, N, 16), jnp.float32),
        scratch_types=[
            pltpu.VMEM((NCH, CH), jnp.int32),
            pltpu.VMEM((NCH, CH), jnp.int32),
            pltpu.VMEM((CH, 16), jnp.float32),
            pltpu.VMEM((CH, 16), jnp.float32),
            pltpu.VMEM_SHARED((N, 16), jnp.float32),
            pltpu.VMEM_SHARED((N, 16), jnp.float32),
            pltpu.SemaphoreType.DMA,
            pltpu.SemaphoreType.DMA,
        ],
    )
    def k(e1, l1, e2, l2, out, idxv, labv, oh0, oh1, cnt1, cnt2, cs0, cs1):
        c = lax.axis_index("c")
        s = lax.axis_index("s")
        wid = s * NCORES + c
        ones16 = jnp.ones((16,), jnp.float32)
        zeros16 = jnp.zeros((16,), jnp.float32)
        iota16 = lax.iota(jnp.int32, 16)
        ohb = (oh0, oh1)
        csem = (cs0, cs1)

        def zero_all(oh):
            def zero_row(j, _):
                oh[j, pl.ds(0, 16)] = zeros16
                return 0

            lax.fori_loop(0, CH, zero_row, 0)

        zero_all(oh0)
        zero_all(oh1)
        for cnt in (cnt1, cnt2):
            for rb in range(RPT // CH):
                pltpu.sync_copy(oh0, cnt.at[pl.ds(s * RPT + rb * CH, CH)])
        plsc.subcore_barrier()

        for e, lb, cnt in ((e1, l1, cnt1), (e2, l2, cnt2)):
            for ept in range(2):
                pltpu.sync_copy(e.at[ept, wid], idxv)
                pltpu.sync_copy(lb.at[ept, wid], labv)

                eights = iota16 * 0 + 8

                def paint(oh, ch, val):
                    for grp in range(CH // 16):
                        lab = labv[ch, pl.ds(grp * 16, 16)]
                        rowi = iota16 + grp * 16
                        plsc.store_scatter(oh, [rowi, lab], val)
                        if ept == 1:
                            plsc.store_scatter(oh, [rowi, eights], val)

                for b in range(2):
                    paint(ohb[b], b, ones16)
                    pltpu.async_copy(ohb[b], cnt.at[idxv.at[b]], csem[b],
                                     add=True)

                def body(i, _):
                    for b in range(2):
                        ch = 2 * i + b
                        pltpu.make_async_copy(
                            ohb[b], cnt.at[idxv.at[ch - 2]], csem[b]).wait()
                        paint(ohb[b], ch - 2, zeros16)
                        paint(ohb[b], ch, ones16)
                        pltpu.async_copy(ohb[b], cnt.at[idxv.at[ch]],
                                         csem[b], add=True)
                    return 0

                lax.fori_loop(1, NCH // 2, body, 0)
                for b in range(2):
                    ch = NCH - 2 + b
                    pltpu.make_async_copy(
                        ohb[b], cnt.at[idxv.at[ch]], csem[b]).wait()
                    paint(ohb[b], ch, zeros16)
        plsc.subcore_barrier()
        for g, cnt in enumerate((cnt1, cnt2)):
            pltpu.sync_copy(cnt.at[pl.ds(s * RPT, RPT)],
                            out.at[c, g, pl.ds(s * RPT, RPT)])

    return k


_sc_agg128 = _make_sc_agg(F0)
_sc_agg64 = _make_sc_agg(F2)
_sc_agg32 = _make_sc_agg(F3)
_sc_cnt = _make_sc_cnt()


def _tc_mid1(part1, cnt, x1, x2, W1, b1, W2, b2):
    def body(pr, cntr, x1r, x2r, w1r, b1r, w2r, b2r, o1r, o2r, degr):
        for g, (xr, outr) in enumerate(((x1r, o1r), (x2r, o2r))):
            deg = cntr[0, g, :, 8] + cntr[1, g, :, 8]
            degc = deg[:, None]
            h1 = jnp.dot(xr[...], w1r[...], preferred_element_type=jnp.float32) + b1r[...]
            aggx = pr[0, g] + pr[1, g]
            agg1 = (jnp.dot(aggx, w1r[...], preferred_element_type=jnp.float32)
                    + degc * b1r[...])
            g1 = jax.nn.relu(agg1 / (degc + 1.0) + h1)
            outr[...] = jnp.dot(g1, w2r[...], preferred_element_type=jnp.float32) + b2r[...]
            degr[g] = deg

    return pl.pallas_call(
        body,
        out_shape=(
            jax.ShapeDtypeStruct((N, F2), jnp.float32),
            jax.ShapeDtypeStruct((N, F2), jnp.float32),
            jax.ShapeDtypeStruct((2, N), jnp.float32),
        ),
    )(part1, cnt, x1, x2, W1, b1.reshape(1, F1), W2, b2.reshape(1, F2))


def _tc_mid2(part2, h2_1, h2_2, deg, W3, b3):
    def body(pr, h1r, h2r, degr, wr, br, o1r, o2r):
        for g, (hr, outr) in enumerate(((h1r, o1r), (h2r, o2r))):
            agg = pr[0, g] + pr[1, g]
            g2 = jax.nn.relu(agg / (degr[g][:, None] + 1.0) + hr[...])
            outr[...] = jnp.dot(g2, wr[...], preferred_element_type=jnp.float32) + br[...]

    return pl.pallas_call(
        body,
        out_shape=(
            jax.ShapeDtypeStruct((N, F3), jnp.float32),
            jax.ShapeDtypeStruct((N, F3), jnp.float32),
        ),
    )(part2, h2_1, h2_2, deg, W3, b3.reshape(1, F3))


KC = 128
WNS_CH = (NPG * NPG) // KC


def _tc_tail(part3, h3_1, h3_2, deg, cnt, Wa, Wt, Wm, bt, Wns, bns,
             Wfc1, bfc1, Wfc2, bfc2, Wsc, bsc):
    def body(pr, h1r, h2r, degr, cntr, war, wtr, wmr, btr, wnsr, bnsr,
             fc1r, bfc1r, fc2r, bfc2r, wscr, bscr, mapr, outr, dv, acc, sc):
        mapv = mapr[...][0]
        hrs = (h1r, h2r)
        c = pl.program_id(0)

        @pl.when(c == 0)
        def _init():
            acc[...] = jnp.zeros((B, F3), jnp.float32)
            for g in range(2):
                agg = pr[0, g] + pr[1, g]
                out3 = agg / (degr[g][:, None] + 1.0) + hrs[g][...]
                dv[g] = jax.nn.relu(out3).reshape(B, NPG, F3)
                sc[g] = jnp.mean(jax.nn.sigmoid(out3))
                cg = cntr[0, g] + cntr[1, g]
                io = lax.broadcasted_iota(jnp.int32, (N, 16), 1)
                nsidx = jnp.max(jnp.where((cg > 0.0) & (io < 8), io, 0), axis=1)
                sum_map_h = jnp.sum(cg * mapv[None, :])
                oh = (io == nsidx[:, None]).astype(jnp.float32)
                sum_map_ns = jnp.sum(oh * mapv[None, :])
                sc[2 + g] = (sum_map_h + 2.0 * sum_map_ns) / (2.0 * (E + N))

        a = dv[0, :, pl.ds(c * (NPG // KC), NPG // KC), :]
        d2 = dv[1]
        sec = lax.dot_general(a, d2, (((2,), (2,)), ((0,), (0,))),
                              preferred_element_type=jnp.float32)
        acc[...] += jnp.dot(sec.reshape(B, WNS_CH), wnsr[...],
                            preferred_element_type=jnp.float32)

        @pl.when(c == KC - 1)
        def _final():
            node_scores = jax.nn.sigmoid(acc[...] + bnsr[...])
            ps = []
            for g in range(2):
                d = dv[g]
                xbar = jnp.mean(d, axis=1)
                ctx = jnp.tanh(jnp.dot(xbar, war[...],
                                       preferred_element_type=jnp.float32))
                sig = jax.nn.sigmoid(
                    lax.dot_general(d, ctx, (((2,), (1,)), ((0,), (0,))),
                                    preferred_element_type=jnp.float32))
                p = lax.dot_general(sig, d, (((1,), (1,)), ((0,), (0,))),
                                    preferred_element_type=jnp.float32)
                ps.append(p)
            p1, p2 = ps
            t = jnp.dot(p1, wtr[...].reshape(F3, F3 * T),
                        preferred_element_type=jnp.float32).reshape(B, F3, T)
            scoring = lax.dot_general(t, p2, (((1,), (1,)), ((0,), (0,))),
                                      preferred_element_type=jnp.float32)
            block = jnp.dot(jnp.concatenate([p1, p2], axis=1), wmr[...],
                            preferred_element_type=jnp.float32)
            gl = jax.nn.relu(scoring + block + btr[...])
            s = jnp.concatenate([gl, node_scores], axis=1)
            s = jax.nn.relu(jnp.dot(s, fc1r[...],
                                    preferred_element_type=jnp.float32) + bfc1r[...])
            s = jax.nn.relu(jnp.dot(s, fc2r[...],
                                    preferred_element_type=jnp.float32) + bfc2r[...])
            score = jax.nn.sigmoid(jnp.dot(s, wscr[...],
                                           preferred_element_type=jnp.float32) + bscr[...])
            sup = sc[0] * sc[2] + sc[1] * sc[3]
            outr[...] = jnp.concatenate(
                [score, jnp.full((B, 1), sup, jnp.float32),
                 jnp.zeros((B, 126), jnp.float32)], axis=1)

    const = pl.BlockSpec(lambda c: tuple([0] * 4))
    return pl.pallas_call(
        body,
        grid=(KC,),
        in_specs=[
            pl.BlockSpec((2, 2, N, F3), lambda c: (0, 0, 0, 0)),
            pl.BlockSpec((N, F3), lambda c: (0, 0)),
            pl.BlockSpec((N, F3), lambda c: (0, 0)),
            pl.BlockSpec((2, N), lambda c: (0, 0)),
            pl.BlockSpec((2, 2, N, 16), lambda c: (0, 0, 0, 0)),
            pl.BlockSpec((F3, F3), lambda c: (0, 0)),
            pl.BlockSpec((F3, F3, T), lambda c: (0, 0, 0)),
            pl.BlockSpec((2 * F3, T), lambda c: (0, 0)),
            pl.BlockSpec((1, T), lambda c: (0, 0)),
            pl.BlockSpec((WNS_CH, F3), lambda c: (c, 0)),
            pl.BlockSpec((1, F3), lambda c: (0, 0)),
            pl.BlockSpec((T + F3, T), lambda c: (0, 0)),
            pl.BlockSpec((1, T), lambda c: (0, 0)),
            pl.BlockSpec((T, 4), lambda c: (0, 0)),
            pl.BlockSpec((1, 4), lambda c: (0, 0)),
            pl.BlockSpec((4, 1), lambda c: (0, 0)),
            pl.BlockSpec((1, 1), lambda c: (0, 0)),
            pl.BlockSpec((1, 16), lambda c: (0, 0)),
        ],
        out_specs=pl.BlockSpec((B, 128), lambda c: (0, 0)),
        out_shape=jax.ShapeDtypeStruct((B, 128), jnp.float32),
        scratch_shapes=[
            pltpu.VMEM((2, B, NPG, F3), jnp.float32),
            pltpu.VMEM((B, F3), jnp.float32),
            pltpu.SMEM((4,), jnp.float32),
        ],
    )(part3, h3_1, h3_2, deg, cnt, Wa, Wt, Wm, bt.reshape(1, T),
      Wns, bns.reshape(1, F3),
      Wfc1, bfc1.reshape(1, T), Wfc2, bfc2.reshape(1, 4),
      Wsc, bsc.reshape(1, 1),
      jnp.array(_MAPV + [0.0] * 8, dtype=jnp.float32).reshape(1, 16))


def kernel(x1, x2, edge_index1, edge_index2, H1, H2, batch1, batch2,
           W1, b1, W2, b2, W3, b3, Wa, Wt, Wm, bt, Wns, bns,
           Wfc1, bfc1, Wfc2, bfc2, Wsc, bsc):
    e1 = edge_index1.reshape(2, NW, NCH, CH)
    e2 = edge_index2.reshape(2, NW, NCH, CH)
    h1l = H1.reshape(2, NW, NCH, CH)
    h2l = H2.reshape(2, NW, NCH, CH)

    cnt = _sc_cnt(e1, h1l, e2, h2l)
    part1 = _sc_agg128(x1, x2, e1, e2)
    h2_1, h2_2, deg = _tc_mid1(part1, cnt, x1, x2, W1, b1, W2, b2)
    part2 = _sc_agg64(h2_1, h2_2, e1, e2)
    h3_1, h3_2 = _tc_mid2(part2, h2_1, h2_2, deg, W3, b3)
    part3 = _sc_agg32(h3_1, h3_2, e1, e2)
    out = _tc_tail(part3, h3_1, h3_2, deg, cnt, Wa, Wt, Wm, bt, Wns, bns,
                   Wfc1, bfc1, Wfc2, bfc2, Wsc, bsc)
    return out[:, :1], out[0, 1]


# speedup vs baseline: 31.7259x; 1.0235x over previous
import functools

import jax
import jax.numpy as jnp
from jax import lax
from jax.experimental import pallas as pl
from jax.experimental.pallas import tpu as pltpu
from jax.experimental.pallas import tpu_sc as plsc

N = 4096
NPG = 512
B = 8
E = 262144
F0 = 128
F1 = 128
F2 = 64
F3 = 32
T = 16
_MAPV = [0.0, 0.12, 0.204, 0.186, 0.244, 0.147, 0.039, 0.057]

NCORES = 2
NTILES = 16
NW = NCORES * NTILES
EW = E // NW
CH = 128
NCH = EW // CH
RPT = N // NTILES


def _mesh():
    return plsc.VectorSubcoreMesh(core_axis_name="c", subcore_axis_name="s")


def _make_sc_agg(Fb, NBUF=4):
    LA = NBUF // 2

    @functools.partial(
        pl.kernel,
        mesh=_mesh(),
        compiler_params=pltpu.CompilerParams(use_tc_tiling_on_sc=False, needs_layout_passes=False),
        out_type=jax.ShapeDtypeStruct((NCORES, 2, N, Fb), jnp.float32),
        scratch_types=[
            pltpu.VMEM((NCH, CH), jnp.int32),
            pltpu.VMEM((NCH, CH), jnp.int32),
        ] + [pltpu.VMEM((CH, Fb), jnp.float32)] * NBUF
          + [pltpu.VMEM_SHARED((N, Fb), jnp.float32)]
          + [pltpu.SemaphoreType.DMA] * (2 * NBUF),
    )
    def k(h1, h2, e1, e2, out, srcv, dstv, *rest):
        bufs = rest[:NBUF]
        agg = rest[NBUF]
        gsems = rest[NBUF + 1:2 * NBUF + 1]
        ssems = rest[2 * NBUF + 1:3 * NBUF + 1]
        rows = bufs[0]
        c = lax.axis_index("c")
        s = lax.axis_index("s")
        wid = s * NCORES + c

        def zero_agg():
            def zero_row(j, _):
                for l in range(Fb // 16):
                    rows[j, pl.ds(l * 16, 16)] = jnp.zeros((16,), jnp.float32)
                return 0

            lax.fori_loop(0, CH, zero_row, 0)
            for rb in range(RPT // CH):
                pltpu.sync_copy(rows, agg.at[pl.ds(s * RPT + rb * CH, CH)])
            plsc.subcore_barrier()

        zero_agg()
        for g, (h, e) in enumerate(((h1, e1), (h2, e2))):
            pltpu.sync_copy(e.at[0, wid], srcv)
            pltpu.sync_copy(e.at[1, wid], dstv)

            gh = [None] * NCH
            sh = [None] * NCH
            for p in range(LA):
                gh[p] = pltpu.async_copy(h.at[srcv.at[p]], bufs[p], gsems[p])
            for ch in range(NCH):
                b = ch % NBUF
                gh[ch].wait()
                sh[ch] = pltpu.async_copy(bufs[b], agg.at[dstv.at[ch]],
                                          ssems[b], add=True)
                if ch >= LA:
                    sh[ch - LA].wait()
                if ch + LA < NCH:
                    nb = (ch + LA) % NBUF
                    gh[ch + LA] = pltpu.async_copy(h.at[srcv.at[ch + LA]],
                                                   bufs[nb], gsems[nb])
            for p in range(LA):
                sh[NCH - LA + p].wait()
            plsc.subcore_barrier()
            pltpu.sync_copy(agg.at[pl.ds(s * RPT, RPT)],
                            out.at[c, g, pl.ds(s * RPT, RPT)])
            if g == 0:
                zero_agg()

    return k


def _make_sc_cnt():
    @functools.partial(
        pl.kernel,
        mesh=_mesh(),
        compiler_params=pltpu.CompilerParams(use_tc_tiling_on_sc=False, needs_layout_passes=False),
        out_type=jax.ShapeDtypeStruct((NCORES, 2, N, 16), jnp.float32),
        scratch_types=[
            pltpu.VMEM((NCH, CH), jnp.int32),
            pltpu.VMEM((NCH, CH), jnp.int32),
            pltpu.VMEM((CH, 16), jnp.float32),
            pltpu.VMEM((CH, 16), jnp.float32),
            pltpu.VMEM_SHARED((N, 16), jnp.float32),
            pltpu.VMEM_SHARED((N, 16), jnp.float32),
            pltpu.SemaphoreType.DMA,
            pltpu.SemaphoreType.DMA,
        ],
    )
    def k(e1, l1, e2, l2, out, idxv, labv, oh0, oh1, cnt1, cnt2, cs0, cs1):
        c = lax.axis_index("c")
        s = lax.axis_index("s")
        wid = s * NCORES + c
        ones16 = jnp.ones((16,), jnp.float32)
        zeros16 = jnp.zeros((16,), jnp.float32)
        iota16 = lax.iota(jnp.int32, 16)
        ohb = (oh0, oh1)
        csem = (cs0, cs1)

        def zero_all(oh):
            def zero_row(j, _):
                oh[j, pl.ds(0, 16)] = zeros16
                return 0

            lax.fori_loop(0, CH, zero_row, 0)

        zero_all(oh0)
        zero_all(oh1)
        for cnt in (cnt1, cnt2):
            for rb in range(RPT // CH):
                pltpu.sync_copy(oh0, cnt.at[pl.ds(s * RPT + rb * CH, CH)])
        plsc.subcore_barrier()

        for e, lb, cnt in ((e1, l1, cnt1), (e2, l2, cnt2)):
            for ept in range(2):
                pltpu.sync_copy(e.at[ept, wid], idxv)
                pltpu.sync_copy(lb.at[ept, wid], labv)

                eights = iota16 * 0 + 8

                def paint(oh, ch, val):
                    for grp in range(CH // 16):
                        lab = labv[ch, pl.ds(grp * 16, 16)]
                        rowi = iota16 + grp * 16
                        plsc.store_scatter(oh, [rowi, lab], val)
                        if ept == 1:
                            plsc.store_scatter(oh, [rowi, eights], val)

                for b in range(2):
                    paint(ohb[b], b, ones16)
                    pltpu.async_copy(ohb[b], cnt.at[idxv.at[b]], csem[b],
                                     add=True)

                def body(i, _):
                    for b in range(2):
                        ch = 2 * i + b
                        pltpu.make_async_copy(
                            ohb[b], cnt.at[idxv.at[ch - 2]], csem[b]).wait()
                        paint(ohb[b], ch - 2, zeros16)
                        paint(ohb[b], ch, ones16)
                        pltpu.async_copy(ohb[b], cnt.at[idxv.at[ch]],
                                         csem[b], add=True)
                    return 0

                lax.fori_loop(1, NCH // 2, body, 0)
                for b in range(2):
                    ch = NCH - 2 + b
                    pltpu.make_async_copy(
                        ohb[b], cnt.at[idxv.at[ch]], csem[b]).wait()
                    paint(ohb[b], ch, zeros16)
        plsc.subcore_barrier()
        for g, cnt in enumerate((cnt1, cnt2)):
            pltpu.sync_copy(cnt.at[pl.ds(s * RPT, RPT)],
                            out.at[c, g, pl.ds(s * RPT, RPT)])

    return k


_sc_agg128 = _make_sc_agg(F0, 4)
_sc_agg64 = _make_sc_agg(F2, 6)
_sc_agg32 = _make_sc_agg(F3, 6)
_sc_cnt = _make_sc_cnt()


def _tc_mid1(part1, cnt, x1, x2, W1, b1, W2, b2):
    def body(pr, cntr, x1r, x2r, w1r, b1r, w2r, b2r, o1r, o2r, degr):
        for g, (xr, outr) in enumerate(((x1r, o1r), (x2r, o2r))):
            deg = cntr[0, g, :, 8] + cntr[1, g, :, 8]
            degc = deg[:, None]
            h1 = jnp.dot(xr[...], w1r[...], preferred_element_type=jnp.float32) + b1r[...]
            aggx = pr[0, g] + pr[1, g]
            agg1 = (jnp.dot(aggx, w1r[...], preferred_element_type=jnp.float32)
                    + degc * b1r[...])
            g1 = jax.nn.relu(agg1 / (degc + 1.0) + h1)
            outr[...] = jnp.dot(g1, w2r[...], preferred_element_type=jnp.float32) + b2r[...]
            degr[g] = deg

    return pl.pallas_call(
        body,
        out_shape=(
            jax.ShapeDtypeStruct((N, F2), jnp.float32),
            jax.ShapeDtypeStruct((N, F2), jnp.float32),
            jax.ShapeDtypeStruct((2, N), jnp.float32),
        ),
    )(part1, cnt, x1, x2, W1, b1.reshape(1, F1), W2, b2.reshape(1, F2))


def _tc_mid2(part2, h2_1, h2_2, deg, W3, b3):
    def body(pr, h1r, h2r, degr, wr, br, o1r, o2r):
        for g, (hr, outr) in enumerate(((h1r, o1r), (h2r, o2r))):
            agg = pr[0, g] + pr[1, g]
            g2 = jax.nn.relu(agg / (degr[g][:, None] + 1.0) + hr[...])
            outr[...] = jnp.dot(g2, wr[...], preferred_element_type=jnp.float32) + br[...]

    return pl.pallas_call(
        body,
        out_shape=(
            jax.ShapeDtypeStruct((N, F3), jnp.float32),
            jax.ShapeDtypeStruct((N, F3), jnp.float32),
        ),
    )(part2, h2_1, h2_2, deg, W3, b3.reshape(1, F3))


KC = 128
WNS_CH = (NPG * NPG) // KC


def _tc_tail(part3, h3_1, h3_2, deg, cnt, Wa, Wt, Wm, bt, Wns, bns,
             Wfc1, bfc1, Wfc2, bfc2, Wsc, bsc):
    def body(pr, h1r, h2r, degr, cntr, war, wtr, wmr, btr, wnsr, bnsr,
             fc1r, bfc1r, fc2r, bfc2r, wscr, bscr, mapr, outr, dv, acc, sc):
        mapv = mapr[...][0]
        hrs = (h1r, h2r)
        c = pl.program_id(0)

        @pl.when(c == 0)
        def _init():
            acc[...] = jnp.zeros((B, F3), jnp.float32)
            for g in range(2):
                agg = pr[0, g] + pr[1, g]
                out3 = agg / (degr[g][:, None] + 1.0) + hrs[g][...]
                dv[g] = jax.nn.relu(out3).reshape(B, NPG, F3)
                sc[g] = jnp.mean(jax.nn.sigmoid(out3))
                cg = cntr[0, g] + cntr[1, g]
                io = lax.broadcasted_iota(jnp.int32, (N, 16), 1)
                nsidx = jnp.max(jnp.where((cg > 0.0) & (io < 8), io, 0), axis=1)
                sum_map_h = jnp.sum(cg * mapv[None, :])
                oh = (io == nsidx[:, None]).astype(jnp.float32)
                sum_map_ns = jnp.sum(oh * mapv[None, :])
                sc[2 + g] = (sum_map_h + 2.0 * sum_map_ns) / (2.0 * (E + N))

        a = dv[0, :, pl.ds(c * (NPG // KC), NPG // KC), :]
        d2 = dv[1]
        sec = lax.dot_general(a, d2, (((2,), (2,)), ((0,), (0,))),
                              preferred_element_type=jnp.float32)
        acc[...] += jnp.dot(sec.reshape(B, WNS_CH), wnsr[...],
                            preferred_element_type=jnp.float32)

        @pl.when(c == KC - 1)
        def _final():
            node_scores = jax.nn.sigmoid(acc[...] + bnsr[...])
            ps = []
            for g in range(2):
                d = dv[g]
                xbar = jnp.mean(d, axis=1)
                ctx = jnp.tanh(jnp.dot(xbar, war[...],
                                       preferred_element_type=jnp.float32))
                sig = jax.nn.sigmoid(
                    lax.dot_general(d, ctx, (((2,), (1,)), ((0,), (0,))),
                                    preferred_element_type=jnp.float32))
                p = lax.dot_general(sig, d, (((1,), (1,)), ((0,), (0,))),
                                    preferred_element_type=jnp.float32)
                ps.append(p)
            p1, p2 = ps
            t = jnp.dot(p1, wtr[...].reshape(F3, F3 * T),
                        preferred_element_type=jnp.float32).reshape(B, F3, T)
            scoring = lax.dot_general(t, p2, (((1,), (1,)), ((0,), (0,))),
                                      preferred_element_type=jnp.float32)
            block = jnp.dot(jnp.concatenate([p1, p2], axis=1), wmr[...],
                            preferred_element_type=jnp.float32)
            gl = jax.nn.relu(scoring + block + btr[...])
            s = jnp.concatenate([gl, node_scores], axis=1)
            s = jax.nn.relu(jnp.dot(s, fc1r[...],
                                    preferred_element_type=jnp.float32) + bfc1r[...])
            s = jax.nn.relu(jnp.dot(s, fc2r[...],
                                    preferred_element_type=jnp.float32) + bfc2r[...])
            score = jax.nn.sigmoid(jnp.dot(s, wscr[...],
                                           preferred_element_type=jnp.float32) + bscr[...])
            sup = sc[0] * sc[2] + sc[1] * sc[3]
            outr[...] = jnp.concatenate(
                [score, jnp.full((B, 1), sup, jnp.float32),
                 jnp.zeros((B, 126), jnp.float32)], axis=1)

    const = pl.BlockSpec(lambda c: tuple([0] * 4))
    return pl.pallas_call(
        body,
        grid=(KC,),
        in_specs=[
            pl.BlockSpec((2, 2, N, F3), lambda c: (0, 0, 0, 0)),
            pl.BlockSpec((N, F3), lambda c: (0, 0)),
            pl.BlockSpec((N, F3), lambda c: (0, 0)),
            pl.BlockSpec((2, N), lambda c: (0, 0)),
            pl.BlockSpec((2, 2, N, 16), lambda c: (0, 0, 0, 0)),
            pl.BlockSpec((F3, F3), lambda c: (0, 0)),
            pl.BlockSpec((F3, F3, T), lambda c: (0, 0, 0)),
            pl.BlockSpec((2 * F3, T), lambda c: (0, 0)),
            pl.BlockSpec((1, T), lambda c: (0, 0)),
            pl.BlockSpec((WNS_CH, F3), lambda c: (c, 0)),
            pl.BlockSpec((1, F3), lambda c: (0, 0)),
            pl.BlockSpec((T + F3, T), lambda c: (0, 0)),
            pl.BlockSpec((1, T), lambda c: (0, 0)),
            pl.BlockSpec((T, 4), lambda c: (0, 0)),
            pl.BlockSpec((1, 4), lambda c: (0, 0)),
            pl.BlockSpec((4, 1), lambda c: (0, 0)),
            pl.BlockSpec((1, 1), lambda c: (0, 0)),
            pl.BlockSpec((1, 16), lambda c: (0, 0)),
        ],
        out_specs=pl.BlockSpec((B, 128), lambda c: (0, 0)),
        out_shape=jax.ShapeDtypeStruct((B, 128), jnp.float32),
        scratch_shapes=[
            pltpu.VMEM((2, B, NPG, F3), jnp.float32),
            pltpu.VMEM((B, F3), jnp.float32),
            pltpu.SMEM((4,), jnp.float32),
        ],
    )(part3, h3_1, h3_2, deg, cnt, Wa, Wt, Wm, bt.reshape(1, T),
      Wns, bns.reshape(1, F3),
      Wfc1, bfc1.reshape(1, T), Wfc2, bfc2.reshape(1, 4),
      Wsc, bsc.reshape(1, 1),
      jnp.array(_MAPV + [0.0] * 8, dtype=jnp.float32).reshape(1, 16))


def kernel(x1, x2, edge_index1, edge_index2, H1, H2, batch1, batch2,
           W1, b1, W2, b2, W3, b3, Wa, Wt, Wm, bt, Wns, bns,
           Wfc1, bfc1, Wfc2, bfc2, Wsc, bsc):
    e1 = edge_index1.reshape(2, NW, NCH, CH)
    e2 = edge_index2.reshape(2, NW, NCH, CH)
    h1l = H1.reshape(2, NW, NCH, CH)
    h2l = H2.reshape(2, NW, NCH, CH)

    cnt = _sc_cnt(e1, h1l, e2, h2l)
    part1 = _sc_agg128(x1, x2, e1, e2)
    h2_1, h2_2, deg = _tc_mid1(part1, cnt, x1, x2, W1, b1, W2, b2)
    part2 = _sc_agg64(h2_1, h2_2, e1, e2)
    h3_1, h3_2 = _tc_mid2(part2, h2_1, h2_2, deg, W3, b3)
    part3 = _sc_agg32(h3_1, h3_2, e1, e2)
    out = _tc_tail(part3, h3_1, h3_2, deg, cnt, Wa, Wt, Wm, bt, Wns, bns,
                   Wfc1, bfc1, Wfc2, bfc2, Wsc, bsc)
    return out[:, :1], out[0, 1]
